# baseline (speedup 1.0000x reference)
import functools

import jax
import jax.numpy as jnp
from jax import lax
from jax.experimental import pallas as pl
from jax.experimental.pallas import tpu as pltpu

HIDDEN = 64
IN_DIM = 128
N_LAYERS = 2
MAX_DIM = 2
B = 128
FHM = 2
EPS = 1e-5
BLK = 2000


def _scale_shift(s, ss, n, g, be):
    m = s / n
    v = ss / n - m * m
    sc = g * lax.rsqrt(v + EPS)
    sh = be - m * sc
    return jnp.stack([sc, sh], axis=0)


def _proj_body(nouts, nb, *refs):
    if len(refs) == 3 + nouts:
        x_ref, ss_ref, w_ref = refs[:3]
        xb = x_ref[...] * ss_ref[0:1, :] + ss_ref[1:2, :]
        xb = jnp.maximum(xb, 0.0)
    else:
        x_ref, w_ref = refs[:2]
        xb = x_ref[...]
    outs = refs[-nouts:]
    y = jnp.dot(xb, w_ref[...], preferred_element_type=jnp.float32)
    for k, o_ref in enumerate(outs):
        o_ref[...] = y[:, k * HIDDEN:(k + 1) * HIDDEN]


def _proj(x, ws, ss=None):
    n, k = x.shape
    nb = n // BLK
    wcat = jnp.concatenate(ws, axis=1)
    nouts = len(ws)
    in_specs = [pl.BlockSpec((BLK, k), lambda i: (i, 0))]
    args = [x]
    if ss is not None:
        in_specs.append(pl.BlockSpec((2, k), lambda i: (0, 0)))
        args.append(ss)
    in_specs.append(pl.BlockSpec((k, nouts * HIDDEN), lambda i: (0, 0)))
    args.append(wcat)
    return pl.pallas_call(
        functools.partial(_proj_body, nouts, nb),
        grid=(nb,),
        in_specs=in_specs,
        out_specs=[pl.BlockSpec((BLK, HIDDEN), lambda i: (i, 0))] * nouts,
        out_shape=[jax.ShapeDtypeStruct((n, HIDDEN), jnp.float32)] * nouts,
    )(*args)


def _p2_body(n, nb, a_ref, g1_ref, w2_ref, g2_ref, z_ref, ss2_ref,
             acc_a, ss1, acc_z):
    s = pl.program_id(0)
    i = pl.program_id(1)
    ab = a_ref[...]

    @pl.when((s == 0) & (i == 0))
    def _():
        acc_a[...] = jnp.zeros_like(acc_a)

    @pl.when(s == 0)
    def _():
        acc_a[...] += jnp.stack(
            [jnp.sum(ab, axis=0), jnp.sum(ab * ab, axis=0)], axis=0)
        z_ref[...] = jnp.zeros_like(z_ref)
        ss2_ref[...] = jnp.zeros_like(ss2_ref)

    @pl.when((s == 1) & (i == 0))
    def _():
        st = acc_a[...]
        ss1[...] = _scale_shift(st[0], st[1], float(n), g1_ref[0], g1_ref[1])
        acc_z[...] = jnp.zeros_like(acc_z)

    @pl.when(s == 1)
    def _():
        h1 = jnp.maximum(ab * ss1[0:1, :] + ss1[1:2, :], 0.0)
        z = jnp.dot(h1, w2_ref[...], preferred_element_type=jnp.float32)
        z_ref[...] = z
        acc_z[...] += jnp.stack(
            [jnp.sum(z, axis=0), jnp.sum(z * z, axis=0)], axis=0)
        st = acc_z[...]
        ss2_ref[...] = _scale_shift(st[0], st[1], float(n), g2_ref[0], g2_ref[1])


def _p2(a, p):
    n = a.shape[0]
    nb = n // BLK
    g1 = jnp.stack([p["g1"], p["be1"]], axis=0)
    g2 = jnp.stack([p["g2"], p["be2"]], axis=0)
    return pl.pallas_call(
        functools.partial(_p2_body, n, nb),
        grid=(2, nb),
        in_specs=[
            pl.BlockSpec((BLK, HIDDEN), lambda s, i: (i, 0)),
            pl.BlockSpec((2, HIDDEN), lambda s, i: (0, 0)),
            pl.BlockSpec((HIDDEN, HIDDEN), lambda s, i: (0, 0)),
            pl.BlockSpec((2, HIDDEN), lambda s, i: (0, 0)),
        ],
        out_specs=[
            pl.BlockSpec((BLK, HIDDEN), lambda s, i: (i, 0)),
            pl.BlockSpec((2, HIDDEN), lambda s, i: (0, 0)),
        ],
        out_shape=[
            jax.ShapeDtypeStruct((n, HIDDEN), jnp.float32),
            jax.ShapeDtypeStruct((2, HIDDEN), jnp.float32),
        ],
        scratch_shapes=[pltpu.VMEM((2, HIDDEN), jnp.float32)] * 3,
    )(a, g1, p["W2"], g2)


def _p3_body(n, nb, zu_ref, ssu_ref, zb_ref, ssb_ref, wc_ref, g3_ref,
             c_ref, ss3_ref, acc_c):
    i = pl.program_id(0)

    @pl.when(i == 0)
    def _():
        acc_c[...] = jnp.zeros_like(acc_c)

    zu = jnp.maximum(zu_ref[...] * ssu_ref[0:1, :] + ssu_ref[1:2, :], 0.0)
    zb = jnp.maximum(zb_ref[...] * ssb_ref[0:1, :] + ssb_ref[1:2, :], 0.0)
    wc = wc_ref[...]
    c = (jnp.dot(zu, wc[:HIDDEN], preferred_element_type=jnp.float32)
         + jnp.dot(zb, wc[HIDDEN:], preferred_element_type=jnp.float32))
    c_ref[...] = c
    acc_c[...] += jnp.stack([jnp.sum(c, axis=0), jnp.sum(c * c, axis=0)], axis=0)
    st = acc_c[...]
    ss3_ref[...] = _scale_shift(st[0], st[1], float(n), g3_ref[0], g3_ref[1])


def _p3(z_up, ss_up, z_bd, ss_bd, pc):
    n = z_up.shape[0]
    nb = n // BLK
    g3 = jnp.stack([pc["g"], pc["be"]], axis=0)
    return pl.pallas_call(
        functools.partial(_p3_body, n, nb),
        grid=(nb,),
        in_specs=[
            pl.BlockSpec((BLK, HIDDEN), lambda i: (i, 0)),
            pl.BlockSpec((2, HIDDEN), lambda i: (0, 0)),
            pl.BlockSpec((BLK, HIDDEN), lambda i: (i, 0)),
            pl.BlockSpec((2, HIDDEN), lambda i: (0, 0)),
            pl.BlockSpec((2 * HIDDEN, HIDDEN), lambda i: (0, 0)),
            pl.BlockSpec((2, HIDDEN), lambda i: (0, 0)),
        ],
        out_specs=[
            pl.BlockSpec((BLK, HIDDEN), lambda i: (i, 0)),
            pl.BlockSpec((2, HIDDEN), lambda i: (0, 0)),
        ],
        out_shape=[
            jax.ShapeDtypeStruct((n, HIDDEN), jnp.float32),
            jax.ShapeDtypeStruct((2, HIDDEN), jnp.float32),
        ],
        scratch_shapes=[pltpu.VMEM((2, HIDDEN), jnp.float32)],
    )(z_up, ss_up, z_bd, ss_bd, pc["W"], g3)


def _p4_body(nb, c_ref, ss_ref, ids_ref, out_ref, acc):
    i = pl.program_id(0)

    @pl.when(i == 0)
    def _():
        acc[...] = jnp.zeros_like(acc)

    cb = jnp.maximum(c_ref[...] * ss_ref[0:1, :] + ss_ref[1:2, :], 0.0)
    ids = ids_ref[0, 0, :]
    onehot_t = (lax.broadcasted_iota(jnp.int32, (B, BLK), 0)
                == ids[None, :]).astype(jnp.float32)
    acc[...] += jnp.dot(onehot_t, cb, preferred_element_type=jnp.float32)
    out_ref[...] = acc[...]


def _p4(c, ss, batch):
    n = c.shape[0]
    nb = n // BLK
    ids3 = batch.reshape(nb, 1, BLK)
    return pl.pallas_call(
        functools.partial(_p4_body, nb),
        grid=(nb,),
        in_specs=[
            pl.BlockSpec((BLK, HIDDEN), lambda i: (i, 0)),
            pl.BlockSpec((2, HIDDEN), lambda i: (0, 0)),
            pl.BlockSpec((1, 1, BLK), lambda i: (i, 0, 0)),
        ],
        out_specs=pl.BlockSpec((B, HIDDEN), lambda i: (0, 0)),
        out_shape=jax.ShapeDtypeStruct((B, HIDDEN), jnp.float32),
        scratch_shapes=[pltpu.VMEM((B, HIDDEN), jnp.float32)],
    )(c, ss, ids3)


def _p5_body(p0, p1, p2, w0, w1, w2, b0, b1, b2, out_ref):
    acc = jnp.zeros((B, FHM * HIDDEN), jnp.float32)
    for p, w, b in ((p0, w0, b0), (p1, w1, b1), (p2, w2, b2)):
        acc += jnp.maximum(
            jnp.dot(p[...], w[...], preferred_element_type=jnp.float32)
            + b[...], 0.0)
    out_ref[...] = acc


def _p5(pooled, lin1):
    args = list(pooled) + [lin1[d]["W"] for d in range(3)] \
        + [lin1[d]["b"].reshape(1, -1) for d in range(3)]
    return pl.pallas_call(
        _p5_body,
        out_shape=jax.ShapeDtypeStruct((B, FHM * HIDDEN), jnp.float32),
    )(*args)


def _segment_sum_sc(table, src, dst, num_segments, init):
    return init + jax.ops.segment_sum(table[src], dst, num_segments=num_segments)


def kernel(x0, x1, x2, up0, up1, b1_src, b1_dst, b2_src, b2_dst,
           batch0, batch1, batch2, params, lin1):
    ns = (x0.shape[0], x1.shape[0], x2.shape[0])
    srcs = [x0, x1, x2]
    sss = [None, None, None]
    for l in range(N_LAYERS):
        pl0, pl1, pl2 = params[l][0], params[l][1], params[l][2]
        yu0, yb0, yb1s = _proj(
            srcs[0], [pl0["up"]["W1"], pl0["bd"]["W1"], pl1["bd"]["W1"]], sss[0])
        yu1, yb1, yb2s = _proj(
            srcs[1], [pl1["up"]["W1"], pl1["bd"]["W1"], pl2["bd"]["W1"]], sss[1])
        yu2, yb2 = _proj(
            srcs[2], [pl2["up"]["W1"], pl2["bd"]["W1"]], sss[2])
        a_up0 = _segment_sum_sc(yu0, up0[0], up0[1], ns[0], yu0)
        a_up1 = _segment_sum_sc(yu1, up1[0], up1[1], ns[1], yu1)
        a_bd1 = _segment_sum_sc(yb1s, b1_src, b1_dst, ns[1], yb1)
        a_bd2 = _segment_sum_sc(yb2s, b2_src, b2_dst, ns[2], yb2)
        new_srcs, new_sss = [], []
        for d, (a_up, a_bd) in enumerate(((a_up0, yb0), (a_up1, a_bd1),
                                          (yu2, a_bd2))):
            z_u, ss_u = _p2(a_up, params[l][d]["up"])
            z_b, ss_b = _p2(a_bd, params[l][d]["bd"])
            c, ss3 = _p3(z_u, ss_u, z_b, ss_b, params[l][d]["comb"])
            new_srcs.append(c)
            new_sss.append(ss3)
        srcs, sss = new_srcs, new_sss
    pooled = [_p4(srcs[d], sss[d], b)
              for d, b in enumerate((batch0, batch1, batch2))]
    return _p5(pooled, lin1)


# speedup vs baseline: 2.2329x; 2.2329x over previous
import functools

import jax
import jax.numpy as jnp
from jax import lax
from jax.experimental import pallas as pl
from jax.experimental.pallas import tpu as pltpu
from jax.experimental.pallas import tpu_sc as plsc

HIDDEN = 64
IN_DIM = 128
N_LAYERS = 2
MAX_DIM = 2
B = 128
FHM = 2
EPS = 1e-5
BLK = 2000


def _scale_shift(s, ss, n, g, be):
    m = s / n
    v = ss / n - m * m
    sc = g * lax.rsqrt(v + EPS)
    sh = be - m * sc
    return jnp.stack([sc, sh], axis=0)


def _proj_body(nouts, nb, *refs):
    if len(refs) == 3 + nouts:
        x_ref, ss_ref, w_ref = refs[:3]
        xb = x_ref[...] * ss_ref[0:1, :] + ss_ref[1:2, :]
        xb = jnp.maximum(xb, 0.0)
    else:
        x_ref, w_ref = refs[:2]
        xb = x_ref[...]
    outs = refs[-nouts:]
    y = jnp.dot(xb, w_ref[...], preferred_element_type=jnp.float32)
    for k, o_ref in enumerate(outs):
        o_ref[...] = y[:, k * HIDDEN:(k + 1) * HIDDEN]


def _proj(x, ws, ss=None):
    n, k = x.shape
    nb = n // BLK
    wcat = jnp.concatenate(ws, axis=1)
    nouts = len(ws)
    in_specs = [pl.BlockSpec((BLK, k), lambda i: (i, 0))]
    args = [x]
    if ss is not None:
        in_specs.append(pl.BlockSpec((2, k), lambda i: (0, 0)))
        args.append(ss)
    in_specs.append(pl.BlockSpec((k, nouts * HIDDEN), lambda i: (0, 0)))
    args.append(wcat)
    return pl.pallas_call(
        functools.partial(_proj_body, nouts, nb),
        grid=(nb,),
        in_specs=in_specs,
        out_specs=[pl.BlockSpec((BLK, HIDDEN), lambda i: (i, 0))] * nouts,
        out_shape=[jax.ShapeDtypeStruct((n, HIDDEN), jnp.float32)] * nouts,
    )(*args)


def _p2_body(n, nb, a_ref, g1_ref, w2_ref, g2_ref, z_ref, ss2_ref,
             acc_a, ss1, acc_z):
    s = pl.program_id(0)
    i = pl.program_id(1)
    ab = a_ref[...]

    @pl.when((s == 0) & (i == 0))
    def _():
        acc_a[...] = jnp.zeros_like(acc_a)

    @pl.when(s == 0)
    def _():
        acc_a[...] += jnp.stack(
            [jnp.sum(ab, axis=0), jnp.sum(ab * ab, axis=0)], axis=0)
        z_ref[...] = jnp.zeros_like(z_ref)
        ss2_ref[...] = jnp.zeros_like(ss2_ref)

    @pl.when((s == 1) & (i == 0))
    def _():
        st = acc_a[...]
        ss1[...] = _scale_shift(st[0], st[1], float(n), g1_ref[0], g1_ref[1])
        acc_z[...] = jnp.zeros_like(acc_z)

    @pl.when(s == 1)
    def _():
        h1 = jnp.maximum(ab * ss1[0:1, :] + ss1[1:2, :], 0.0)
        z = jnp.dot(h1, w2_ref[...], preferred_element_type=jnp.float32)
        z_ref[...] = z
        acc_z[...] += jnp.stack(
            [jnp.sum(z, axis=0), jnp.sum(z * z, axis=0)], axis=0)
        st = acc_z[...]
        ss2_ref[...] = _scale_shift(st[0], st[1], float(n), g2_ref[0], g2_ref[1])


def _p2(a, p):
    n = a.shape[0]
    nb = n // BLK
    g1 = jnp.stack([p["g1"], p["be1"]], axis=0)
    g2 = jnp.stack([p["g2"], p["be2"]], axis=0)
    return pl.pallas_call(
        functools.partial(_p2_body, n, nb),
        grid=(2, nb),
        in_specs=[
            pl.BlockSpec((BLK, HIDDEN), lambda s, i: (i, 0)),
            pl.BlockSpec((2, HIDDEN), lambda s, i: (0, 0)),
            pl.BlockSpec((HIDDEN, HIDDEN), lambda s, i: (0, 0)),
            pl.BlockSpec((2, HIDDEN), lambda s, i: (0, 0)),
        ],
        out_specs=[
            pl.BlockSpec((BLK, HIDDEN), lambda s, i: (i, 0)),
            pl.BlockSpec((2, HIDDEN), lambda s, i: (0, 0)),
        ],
        out_shape=[
            jax.ShapeDtypeStruct((n, HIDDEN), jnp.float32),
            jax.ShapeDtypeStruct((2, HIDDEN), jnp.float32),
        ],
        scratch_shapes=[pltpu.VMEM((2, HIDDEN), jnp.float32)] * 3,
    )(a, g1, p["W2"], g2)


def _p3_body(n, nb, zu_ref, ssu_ref, zb_ref, ssb_ref, wc_ref, g3_ref,
             c_ref, ss3_ref, acc_c):
    i = pl.program_id(0)

    @pl.when(i == 0)
    def _():
        acc_c[...] = jnp.zeros_like(acc_c)

    zu = jnp.maximum(zu_ref[...] * ssu_ref[0:1, :] + ssu_ref[1:2, :], 0.0)
    zb = jnp.maximum(zb_ref[...] * ssb_ref[0:1, :] + ssb_ref[1:2, :], 0.0)
    wc = wc_ref[...]
    c = (jnp.dot(zu, wc[:HIDDEN], preferred_element_type=jnp.float32)
         + jnp.dot(zb, wc[HIDDEN:], preferred_element_type=jnp.float32))
    c_ref[...] = c
    acc_c[...] += jnp.stack([jnp.sum(c, axis=0), jnp.sum(c * c, axis=0)], axis=0)
    st = acc_c[...]
    ss3_ref[...] = _scale_shift(st[0], st[1], float(n), g3_ref[0], g3_ref[1])


def _p3(z_up, ss_up, z_bd, ss_bd, pc):
    n = z_up.shape[0]
    nb = n // BLK
    g3 = jnp.stack([pc["g"], pc["be"]], axis=0)
    return pl.pallas_call(
        functools.partial(_p3_body, n, nb),
        grid=(nb,),
        in_specs=[
            pl.BlockSpec((BLK, HIDDEN), lambda i: (i, 0)),
            pl.BlockSpec((2, HIDDEN), lambda i: (0, 0)),
            pl.BlockSpec((BLK, HIDDEN), lambda i: (i, 0)),
            pl.BlockSpec((2, HIDDEN), lambda i: (0, 0)),
            pl.BlockSpec((2 * HIDDEN, HIDDEN), lambda i: (0, 0)),
            pl.BlockSpec((2, HIDDEN), lambda i: (0, 0)),
        ],
        out_specs=[
            pl.BlockSpec((BLK, HIDDEN), lambda i: (i, 0)),
            pl.BlockSpec((2, HIDDEN), lambda i: (0, 0)),
        ],
        out_shape=[
            jax.ShapeDtypeStruct((n, HIDDEN), jnp.float32),
            jax.ShapeDtypeStruct((2, HIDDEN), jnp.float32),
        ],
        scratch_shapes=[pltpu.VMEM((2, HIDDEN), jnp.float32)],
    )(z_up, ss_up, z_bd, ss_bd, pc["W"], g3)


def _p4_body(nb, c_ref, ss_ref, ids_ref, out_ref, acc):
    i = pl.program_id(0)

    @pl.when(i == 0)
    def _():
        acc[...] = jnp.zeros_like(acc)

    cb = jnp.maximum(c_ref[...] * ss_ref[0:1, :] + ss_ref[1:2, :], 0.0)
    ids = ids_ref[0, 0, :]
    onehot_t = (lax.broadcasted_iota(jnp.int32, (B, BLK), 0)
                == ids[None, :]).astype(jnp.float32)
    acc[...] += jnp.dot(onehot_t, cb, preferred_element_type=jnp.float32)
    out_ref[...] = acc[...]


def _p4(c, ss, batch):
    n = c.shape[0]
    nb = n // BLK
    ids3 = batch.reshape(nb, 1, BLK)
    return pl.pallas_call(
        functools.partial(_p4_body, nb),
        grid=(nb,),
        in_specs=[
            pl.BlockSpec((BLK, HIDDEN), lambda i: (i, 0)),
            pl.BlockSpec((2, HIDDEN), lambda i: (0, 0)),
            pl.BlockSpec((1, 1, BLK), lambda i: (i, 0, 0)),
        ],
        out_specs=pl.BlockSpec((B, HIDDEN), lambda i: (0, 0)),
        out_shape=jax.ShapeDtypeStruct((B, HIDDEN), jnp.float32),
        scratch_shapes=[pltpu.VMEM((B, HIDDEN), jnp.float32)],
    )(c, ss, ids3)


def _p5_body(p0, p1, p2, w0, w1, w2, b0, b1, b2, out_ref):
    acc = jnp.zeros((B, FHM * HIDDEN), jnp.float32)
    for p, w, b in ((p0, w0, b0), (p1, w1, b1), (p2, w2, b2)):
        acc += jnp.maximum(
            jnp.dot(p[...], w[...], preferred_element_type=jnp.float32)
            + b[...], 0.0)
    out_ref[...] = acc


def _p5(pooled, lin1):
    args = list(pooled) + [lin1[d]["W"] for d in range(3)] \
        + [lin1[d]["b"].reshape(1, -1) for d in range(3)]
    return pl.pallas_call(
        _p5_body,
        out_shape=jax.ShapeDtypeStruct((B, FHM * HIDDEN), jnp.float32),
    )(*args)


_NTILES = 16
_GRP = 128
_W = 2000
_CMAX = 13344


def _seg_cfg(m):
    nch = -(-m // _CMAX)
    nch += nch % 2
    nch = max(2, nch)
    c = ((-(-m // nch)) + 15) // 16 * 16
    tail = m - (nch - 1) * c
    assert 0 < tail <= c and tail % 16 == 0 and c <= _CMAX
    return nch, c, tail


def _emit_segsum(core, tid, lane, table_h, src_h, dst_h, init_h, out_h,
                 win_src, win_dst, comp_src, comp_dst, ldst_stage, rows_v,
                 accum, sem, e, m):
    nch, c, tail = _seg_cfg(m)
    npass = nch // 2
    epw = e // _NTILES
    nwin = epw // _W
    assert epw % _W == 0
    rpt, rpt_t = c // _NTILES, tail // _NTILES
    estart = tid * epw
    trash_src = lane * 8
    trash_dst = _CMAX + (lane & 7)

    for p in range(npass):
        chunk = 2 * p + core
        lo = chunk * c
        last = p == npass - 1

        if last:
            @pl.when(core == 0)
            def _():
                pltpu.sync_copy(init_h.at[pl.ds(lo + tid * rpt, rpt)],
                                accum.at[pl.ds(tid * rpt, rpt)])

            @pl.when(core == 1)
            def _():
                pltpu.sync_copy(init_h.at[pl.ds(lo + tid * rpt_t, rpt_t)],
                                accum.at[pl.ds(tid * rpt_t, rpt_t)])
        else:
            pltpu.sync_copy(init_h.at[pl.ds(lo + tid * rpt, rpt)],
                            accum.at[pl.ds(tid * rpt, rpt)])
        plsc.subcore_barrier()

        def win_body(wi, cur):
            base = estart + wi * _W
            pltpu.sync_copy(src_h.at[pl.ds(base, _W)], win_src)
            pltpu.sync_copy(dst_h.at[pl.ds(base, _W)], win_dst)

            def vbody(v, cur):
                off = v * 16
                dstv = win_dst[pl.ds(off, 16)]
                srcv = win_src[pl.ds(off, 16)]
                local = dstv - lo
                mask = (local >= 0) & (local < c)
                mi = mask.astype(jnp.int32)
                pos = cur + plsc.cumsum(mi) - 1
                plsc.store_scatter(comp_src, [pos], srcv, mask=mask)
                plsc.store_scatter(comp_dst, [pos], local, mask=mask)
                return cur + jnp.sum(mi)

            return lax.fori_loop(0, _W // 16, vbody, cur)

        cursor = lax.fori_loop(0, nwin, win_body, jnp.int32(0))

        for k in range(_GRP // 16):
            posk = cursor + k * 16 + lane
            full = posk >= 0
            plsc.store_scatter(comp_src, [posk], trash_src, mask=full)
            plsc.store_scatter(comp_dst, [posk], trash_dst, mask=full)
        ngroups = (cursor + (_GRP - 1)) >> 7

        def gbody(g, carry):
            base = g * _GRP
            pltpu.async_copy(
                table_h.at[comp_src.at[pl.ds(base, _GRP)]],
                rows_v, sem).wait()
            for k in range(_GRP // 16):
                ldst_stage[pl.ds(k * 16, 16)] = \
                    comp_dst[pl.ds(base + k * 16, 16)]
            pltpu.sync_copy(rows_v, accum.at[ldst_stage], add=True)
            return carry

        lax.fori_loop(0, ngroups, gbody, jnp.int32(0))
        plsc.subcore_barrier()

        if last:
            @pl.when(core == 0)
            def _():
                pltpu.sync_copy(accum.at[pl.ds(tid * rpt, rpt)],
                                out_h.at[pl.ds(lo + tid * rpt, rpt)])

            @pl.when(core == 1)
            def _():
                pltpu.sync_copy(accum.at[pl.ds(tid * rpt_t, rpt_t)],
                                out_h.at[pl.ds(lo + tid * rpt_t, rpt_t)])
        else:
            pltpu.sync_copy(accum.at[pl.ds(tid * rpt, rpt)],
                            out_h.at[pl.ds(lo + tid * rpt, rpt)])
        plsc.subcore_barrier()


def _layer_segsums_sc(yu0, up0s, up0d, n0, yu1, up1s, up1d, n1,
                      yb1s, b1s, b1d, yb1self, yb2s, b2s, b2d, yb2self, n2):
    h = HIDDEN
    jobs = [
        (yu0, up0s, up0d, yu0, up0s.shape[0], n0),
        (yu1, up1s, up1d, yu1, up1s.shape[0], n1),
        (yb1s, b1s, b1d, yb1self, b1s.shape[0], n1),
        (yb2s, b2s, b2d, yb2self, b2s.shape[0], n2),
    ]
    emax = max(j[4] for j in jobs)
    cap = emax // _NTILES + _GRP
    mesh = plsc.VectorSubcoreMesh(core_axis_name="c", subcore_axis_name="s")

    def body(yu0_h, up0s_h, up0d_h, yu1_h, up1s_h, up1d_h,
             yb1s_h, b1s_h, b1d_h, yb1i_h, yb2s_h, b2s_h, b2d_h, yb2i_h,
             o0_h, o1_h, o2_h, o3_h,
             win_src, win_dst, comp_src, comp_dst, ldst_stage, rows_v,
             accum, sem):
        core = lax.axis_index("c")
        tid = lax.axis_index("s")
        lane = lax.iota(jnp.int32, 16)
        tabs = (yu0_h, yu1_h, yb1s_h, yb2s_h)
        srcs = (up0s_h, up1s_h, b1s_h, b2s_h)
        dsts = (up0d_h, up1d_h, b1d_h, b2d_h)
        inits = (yu0_h, yu1_h, yb1i_h, yb2i_h)
        outs = (o0_h, o1_h, o2_h, o3_h)
        for j, (_, _, _, _, e, m) in enumerate(jobs):
            _emit_segsum(core, tid, lane, tabs[j], srcs[j], dsts[j],
                         inits[j], outs[j], win_src, win_dst, comp_src,
                         comp_dst, ldst_stage, rows_v, accum, sem, e, m)

    run = pl.kernel(
        body,
        out_type=[jax.ShapeDtypeStruct((n0, h), jnp.float32),
                  jax.ShapeDtypeStruct((n1, h), jnp.float32),
                  jax.ShapeDtypeStruct((n1, h), jnp.float32),
                  jax.ShapeDtypeStruct((n2, h), jnp.float32)],
        mesh=mesh,
        compiler_params=pltpu.CompilerParams(
            needs_layout_passes=False, use_tc_tiling_on_sc=False),
        scratch_types=[
            pltpu.VMEM((_W,), jnp.int32),
            pltpu.VMEM((_W,), jnp.int32),
            pltpu.VMEM((cap,), jnp.int32),
            pltpu.VMEM((cap,), jnp.int32),
            pltpu.VMEM((_GRP,), jnp.int32),
            pltpu.VMEM((_GRP, h), jnp.float32),
            pltpu.VMEM_SHARED((_CMAX + 8, h), jnp.float32),
            pltpu.SemaphoreType.DMA,
        ],
    )
    return run(yu0, up0s, up0d, yu1, up1s, up1d,
               yb1s, b1s, b1d, yb1self, yb2s, b2s, b2d, yb2self)


def kernel(x0, x1, x2, up0, up1, b1_src, b1_dst, b2_src, b2_dst,
           batch0, batch1, batch2, params, lin1):
    ns = (x0.shape[0], x1.shape[0], x2.shape[0])
    srcs = [x0, x1, x2]
    sss = [None, None, None]
    for l in range(N_LAYERS):
        pl0, pl1, pl2 = params[l][0], params[l][1], params[l][2]
        yu0, yb0, yb1s = _proj(
            srcs[0], [pl0["up"]["W1"], pl0["bd"]["W1"], pl1["bd"]["W1"]], sss[0])
        yu1, yb1, yb2s = _proj(
            srcs[1], [pl1["up"]["W1"], pl1["bd"]["W1"], pl2["bd"]["W1"]], sss[1])
        yu2, yb2 = _proj(
            srcs[2], [pl2["up"]["W1"], pl2["bd"]["W1"]], sss[2])
        a_up0, a_up1, a_bd1, a_bd2 = _layer_segsums_sc(
            yu0, up0[0], up0[1], ns[0], yu1, up1[0], up1[1], ns[1],
            yb1s, b1_src, b1_dst, yb1, yb2s, b2_src, b2_dst, yb2, ns[2])
        new_srcs, new_sss = [], []
        for d, (a_up, a_bd) in enumerate(((a_up0, yb0), (a_up1, a_bd1),
                                          (yu2, a_bd2))):
            z_u, ss_u = _p2(a_up, params[l][d]["up"])
            z_b, ss_b = _p2(a_bd, params[l][d]["bd"])
            c, ss3 = _p3(z_u, ss_u, z_b, ss_b, params[l][d]["comb"])
            new_srcs.append(c)
            new_sss.append(ss3)
        srcs, sss = new_srcs, new_sss
    pooled = [_p4(srcs[d], sss[d], b)
              for d, b in enumerate((batch0, batch1, batch2))]
    return _p5(pooled, lin1)


# speedup vs baseline: 2.4590x; 1.1013x over previous
import functools

import jax
import jax.numpy as jnp
from jax import lax
from jax.experimental import pallas as pl
from jax.experimental.pallas import tpu as pltpu
from jax.experimental.pallas import tpu_sc as plsc

HIDDEN = 64
IN_DIM = 128
N_LAYERS = 2
MAX_DIM = 2
B = 128
FHM = 2
EPS = 1e-5
BLK = 2000


def _scale_shift(s, ss, n, g, be):
    m = s / n
    v = ss / n - m * m
    sc = g * lax.rsqrt(v + EPS)
    sh = be - m * sc
    return jnp.stack([sc, sh], axis=0)


def _proj_body(nouts, nb, *refs):
    if len(refs) == 3 + nouts:
        x_ref, ss_ref, w_ref = refs[:3]
        xb = x_ref[...] * ss_ref[0:1, :] + ss_ref[1:2, :]
        xb = jnp.maximum(xb, 0.0)
    else:
        x_ref, w_ref = refs[:2]
        xb = x_ref[...]
    outs = refs[-nouts:]
    y = jnp.dot(xb, w_ref[...], preferred_element_type=jnp.float32)
    for k, o_ref in enumerate(outs):
        o_ref[...] = y[:, k * HIDDEN:(k + 1) * HIDDEN]


def _proj(x, ws, ss=None):
    n, k = x.shape
    nb = n // BLK
    wcat = jnp.concatenate(ws, axis=1)
    nouts = len(ws)
    in_specs = [pl.BlockSpec((BLK, k), lambda i: (i, 0))]
    args = [x]
    if ss is not None:
        in_specs.append(pl.BlockSpec((2, k), lambda i: (0, 0)))
        args.append(ss)
    in_specs.append(pl.BlockSpec((k, nouts * HIDDEN), lambda i: (0, 0)))
    args.append(wcat)
    return pl.pallas_call(
        functools.partial(_proj_body, nouts, nb),
        grid=(nb,),
        in_specs=in_specs,
        out_specs=[pl.BlockSpec((BLK, HIDDEN), lambda i: (i, 0))] * nouts,
        out_shape=[jax.ShapeDtypeStruct((n, HIDDEN), jnp.float32)] * nouts,
    )(*args)


def _p2_body(n, nb, a_ref, g1_ref, w2_ref, g2_ref, z_ref, ss2_ref,
             acc_a, ss1, acc_z):
    s = pl.program_id(0)
    i = pl.program_id(1)
    ab = a_ref[...]

    @pl.when((s == 0) & (i == 0))
    def _():
        acc_a[...] = jnp.zeros_like(acc_a)

    @pl.when(s == 0)
    def _():
        acc_a[...] += jnp.stack(
            [jnp.sum(ab, axis=0), jnp.sum(ab * ab, axis=0)], axis=0)
        z_ref[...] = jnp.zeros_like(z_ref)
        ss2_ref[...] = jnp.zeros_like(ss2_ref)

    @pl.when((s == 1) & (i == 0))
    def _():
        st = acc_a[...]
        ss1[...] = _scale_shift(st[0], st[1], float(n), g1_ref[0], g1_ref[1])
        acc_z[...] = jnp.zeros_like(acc_z)

    @pl.when(s == 1)
    def _():
        h1 = jnp.maximum(ab * ss1[0:1, :] + ss1[1:2, :], 0.0)
        z = jnp.dot(h1, w2_ref[...], preferred_element_type=jnp.float32)
        z_ref[...] = z
        acc_z[...] += jnp.stack(
            [jnp.sum(z, axis=0), jnp.sum(z * z, axis=0)], axis=0)
        st = acc_z[...]
        ss2_ref[...] = _scale_shift(st[0], st[1], float(n), g2_ref[0], g2_ref[1])


def _p2(a, p):
    n = a.shape[0]
    nb = n // BLK
    g1 = jnp.stack([p["g1"], p["be1"]], axis=0)
    g2 = jnp.stack([p["g2"], p["be2"]], axis=0)
    return pl.pallas_call(
        functools.partial(_p2_body, n, nb),
        grid=(2, nb),
        in_specs=[
            pl.BlockSpec((BLK, HIDDEN), lambda s, i: (i, 0)),
            pl.BlockSpec((2, HIDDEN), lambda s, i: (0, 0)),
            pl.BlockSpec((HIDDEN, HIDDEN), lambda s, i: (0, 0)),
            pl.BlockSpec((2, HIDDEN), lambda s, i: (0, 0)),
        ],
        out_specs=[
            pl.BlockSpec((BLK, HIDDEN), lambda s, i: (i, 0)),
            pl.BlockSpec((2, HIDDEN), lambda s, i: (0, 0)),
        ],
        out_shape=[
            jax.ShapeDtypeStruct((n, HIDDEN), jnp.float32),
            jax.ShapeDtypeStruct((2, HIDDEN), jnp.float32),
        ],
        scratch_shapes=[pltpu.VMEM((2, HIDDEN), jnp.float32)] * 3,
    )(a, g1, p["W2"], g2)


def _p3_body(n, nb, zu_ref, ssu_ref, zb_ref, ssb_ref, wc_ref, g3_ref,
             c_ref, ss3_ref, acc_c):
    i = pl.program_id(0)

    @pl.when(i == 0)
    def _():
        acc_c[...] = jnp.zeros_like(acc_c)

    zu = jnp.maximum(zu_ref[...] * ssu_ref[0:1, :] + ssu_ref[1:2, :], 0.0)
    zb = jnp.maximum(zb_ref[...] * ssb_ref[0:1, :] + ssb_ref[1:2, :], 0.0)
    wc = wc_ref[...]
    c = (jnp.dot(zu, wc[:HIDDEN], preferred_element_type=jnp.float32)
         + jnp.dot(zb, wc[HIDDEN:], preferred_element_type=jnp.float32))
    c_ref[...] = c
    acc_c[...] += jnp.stack([jnp.sum(c, axis=0), jnp.sum(c * c, axis=0)], axis=0)
    st = acc_c[...]
    ss3_ref[...] = _scale_shift(st[0], st[1], float(n), g3_ref[0], g3_ref[1])


def _p3(z_up, ss_up, z_bd, ss_bd, pc):
    n = z_up.shape[0]
    nb = n // BLK
    g3 = jnp.stack([pc["g"], pc["be"]], axis=0)
    return pl.pallas_call(
        functools.partial(_p3_body, n, nb),
        grid=(nb,),
        in_specs=[
            pl.BlockSpec((BLK, HIDDEN), lambda i: (i, 0)),
            pl.BlockSpec((2, HIDDEN), lambda i: (0, 0)),
            pl.BlockSpec((BLK, HIDDEN), lambda i: (i, 0)),
            pl.BlockSpec((2, HIDDEN), lambda i: (0, 0)),
            pl.BlockSpec((2 * HIDDEN, HIDDEN), lambda i: (0, 0)),
            pl.BlockSpec((2, HIDDEN), lambda i: (0, 0)),
        ],
        out_specs=[
            pl.BlockSpec((BLK, HIDDEN), lambda i: (i, 0)),
            pl.BlockSpec((2, HIDDEN), lambda i: (0, 0)),
        ],
        out_shape=[
            jax.ShapeDtypeStruct((n, HIDDEN), jnp.float32),
            jax.ShapeDtypeStruct((2, HIDDEN), jnp.float32),
        ],
        scratch_shapes=[pltpu.VMEM((2, HIDDEN), jnp.float32)],
    )(z_up, ss_up, z_bd, ss_bd, pc["W"], g3)


def _p4_body(nb, c_ref, ss_ref, ids_ref, out_ref, acc):
    i = pl.program_id(0)

    @pl.when(i == 0)
    def _():
        acc[...] = jnp.zeros_like(acc)

    cb = jnp.maximum(c_ref[...] * ss_ref[0:1, :] + ss_ref[1:2, :], 0.0)
    ids = ids_ref[0, 0, :]
    onehot_t = (lax.broadcasted_iota(jnp.int32, (B, BLK), 0)
                == ids[None, :]).astype(jnp.float32)
    acc[...] += jnp.dot(onehot_t, cb, preferred_element_type=jnp.float32)
    out_ref[...] = acc[...]


def _p4(c, ss, batch):
    n = c.shape[0]
    nb = n // BLK
    ids3 = batch.reshape(nb, 1, BLK)
    return pl.pallas_call(
        functools.partial(_p4_body, nb),
        grid=(nb,),
        in_specs=[
            pl.BlockSpec((BLK, HIDDEN), lambda i: (i, 0)),
            pl.BlockSpec((2, HIDDEN), lambda i: (0, 0)),
            pl.BlockSpec((1, 1, BLK), lambda i: (i, 0, 0)),
        ],
        out_specs=pl.BlockSpec((B, HIDDEN), lambda i: (0, 0)),
        out_shape=jax.ShapeDtypeStruct((B, HIDDEN), jnp.float32),
        scratch_shapes=[pltpu.VMEM((B, HIDDEN), jnp.float32)],
    )(c, ss, ids3)


def _p5_body(p0, p1, p2, w0, w1, w2, b0, b1, b2, out_ref):
    acc = jnp.zeros((B, FHM * HIDDEN), jnp.float32)
    for p, w, b in ((p0, w0, b0), (p1, w1, b1), (p2, w2, b2)):
        acc += jnp.maximum(
            jnp.dot(p[...], w[...], preferred_element_type=jnp.float32)
            + b[...], 0.0)
    out_ref[...] = acc


def _p5(pooled, lin1):
    args = list(pooled) + [lin1[d]["W"] for d in range(3)] \
        + [lin1[d]["b"].reshape(1, -1) for d in range(3)]
    return pl.pallas_call(
        _p5_body,
        out_shape=jax.ShapeDtypeStruct((B, FHM * HIDDEN), jnp.float32),
    )(*args)


_NTILES = 16
_GRP = 128
_W = 2000
_CMAX = 12480


def _seg_cfg(m):
    nch = -(-m // _CMAX)
    nch += nch % 2
    nch = max(2, nch)
    c = ((-(-m // nch)) + 15) // 16 * 16
    tail = m - (nch - 1) * c
    assert 0 < tail <= c and tail % 16 == 0 and c <= _CMAX
    return nch, c, tail


_UNROLL = 5


def _emit_segsum(core, tid, lane, table_h, src_h, dst_h, init_h, out_h,
                 win_src, win_dst, comp_src, comp_dst, rows_v0, rows_v1,
                 accum, sem0, sem1, e, m):
    nch, c, tail = _seg_cfg(m)
    npass = nch // 2
    epw = e // _NTILES
    nwin = epw // _W
    assert epw % _W == 0
    rpt, rpt_t = c // _NTILES, tail // _NTILES
    estart = tid * epw
    trash_src = lane * 8
    trash_dst = _CMAX + (lane & 7)

    for p in range(npass):
        chunk = 2 * p + core
        lo = chunk * c
        last = p == npass - 1

        if last:
            @pl.when(core == 0)
            def _():
                pltpu.sync_copy(init_h.at[pl.ds(lo + tid * rpt, rpt)],
                                accum.at[pl.ds(tid * rpt, rpt)])

            @pl.when(core == 1)
            def _():
                pltpu.sync_copy(init_h.at[pl.ds(lo + tid * rpt_t, rpt_t)],
                                accum.at[pl.ds(tid * rpt_t, rpt_t)])
        else:
            pltpu.sync_copy(init_h.at[pl.ds(lo + tid * rpt, rpt)],
                            accum.at[pl.ds(tid * rpt, rpt)])
        plsc.subcore_barrier()

        def win_body(wi, cur):
            base = estart + wi * _W
            pltpu.sync_copy(src_h.at[pl.ds(base, _W)], win_src)
            pltpu.sync_copy(dst_h.at[pl.ds(base, _W)], win_dst)

            def vbody(v, cur):
                off0 = v * (16 * _UNROLL)
                ms, cums, cnts, srcs, locs = [], [], [], [], []
                for k in range(_UNROLL):
                    off = off0 + k * 16
                    dstv = win_dst[pl.ds(off, 16)]
                    srcv = win_src[pl.ds(off, 16)]
                    local = dstv - lo
                    mask = (local >= 0) & (local < c)
                    mi = mask.astype(jnp.int32)
                    ms.append(mask)
                    cums.append(plsc.cumsum(mi))
                    cnts.append(jnp.sum(mi))
                    srcs.append(srcv)
                    locs.append(local)
                base_k = cur
                for k in range(_UNROLL):
                    pos = base_k + cums[k] - 1
                    plsc.store_scatter(comp_src, [pos], srcs[k], mask=ms[k])
                    plsc.store_scatter(
                        comp_dst, [pos >> 7, pos & 127], locs[k], mask=ms[k])
                    base_k = base_k + cnts[k]
                return base_k

            return lax.fori_loop(0, _W // (16 * _UNROLL), vbody, cur)

        cursor = lax.fori_loop(0, nwin, win_body, jnp.int32(0))

        for k in range(_GRP // 16):
            posk = cursor + k * 16 + lane
            full = posk >= 0
            plsc.store_scatter(comp_src, [posk], trash_src, mask=full)
            plsc.store_scatter(
                comp_dst, [posk >> 7, posk & 127], trash_dst, mask=full)
        ngroups = (cursor + (_GRP - 1)) >> 7

        def _issue(g, rows_v, sem):
            pltpu.async_copy(
                table_h.at[comp_src.at[pl.ds(g * _GRP, _GRP)]], rows_v, sem)

        def _wait(rows_v, sem):
            pltpu.make_async_copy(
                table_h.at[comp_src.at[pl.ds(0, _GRP)]], rows_v, sem).wait()

        @pl.when(ngroups > 0)
        def _():
            _issue(0, rows_v0, sem0)

        def gbody(k2, carry):
            g0 = k2 * 2

            @pl.when(g0 < ngroups)
            def _():
                _wait(rows_v0, sem0)

                @pl.when(g0 + 1 < ngroups)
                def _():
                    _issue(g0 + 1, rows_v1, sem1)
                pltpu.sync_copy(rows_v0, accum.at[comp_dst.at[g0]], add=True)

            @pl.when(g0 + 1 < ngroups)
            def _():
                _wait(rows_v1, sem1)

                @pl.when(g0 + 2 < ngroups)
                def _():
                    _issue(g0 + 2, rows_v0, sem0)
                pltpu.sync_copy(rows_v1, accum.at[comp_dst.at[g0 + 1]],
                                add=True)
            return carry

        lax.fori_loop(0, (ngroups + 1) >> 1, gbody, jnp.int32(0))
        plsc.subcore_barrier()

        if last:
            @pl.when(core == 0)
            def _():
                pltpu.sync_copy(accum.at[pl.ds(tid * rpt, rpt)],
                                out_h.at[pl.ds(lo + tid * rpt, rpt)])

            @pl.when(core == 1)
            def _():
                pltpu.sync_copy(accum.at[pl.ds(tid * rpt_t, rpt_t)],
                                out_h.at[pl.ds(lo + tid * rpt_t, rpt_t)])
        else:
            pltpu.sync_copy(accum.at[pl.ds(tid * rpt, rpt)],
                            out_h.at[pl.ds(lo + tid * rpt, rpt)])
        plsc.subcore_barrier()


def _layer_segsums_sc(yu0, up0s, up0d, n0, yu1, up1s, up1d, n1,
                      yb1s, b1s, b1d, yb1self, yb2s, b2s, b2d, yb2self, n2):
    h = HIDDEN
    jobs = [
        (yu0, up0s, up0d, yu0, up0s.shape[0], n0),
        (yu1, up1s, up1d, yu1, up1s.shape[0], n1),
        (yb1s, b1s, b1d, yb1self, b1s.shape[0], n1),
        (yb2s, b2s, b2d, yb2self, b2s.shape[0], n2),
    ]
    emax = max(j[4] for j in jobs)
    capg = emax // _NTILES // _GRP + 2
    cap = capg * _GRP
    mesh = plsc.VectorSubcoreMesh(core_axis_name="c", subcore_axis_name="s")

    def body(yu0_h, up0s_h, up0d_h, yu1_h, up1s_h, up1d_h,
             yb1s_h, b1s_h, b1d_h, yb1i_h, yb2s_h, b2s_h, b2d_h, yb2i_h,
             o0_h, o1_h, o2_h, o3_h,
             win_src, win_dst, comp_src, comp_dst, rows_v0, rows_v1,
             accum, sem0, sem1):
        core = lax.axis_index("c")
        tid = lax.axis_index("s")
        lane = lax.iota(jnp.int32, 16)
        tabs = (yu0_h, yu1_h, yb1s_h, yb2s_h)
        srcs = (up0s_h, up1s_h, b1s_h, b2s_h)
        dsts = (up0d_h, up1d_h, b1d_h, b2d_h)
        inits = (yu0_h, yu1_h, yb1i_h, yb2i_h)
        outs = (o0_h, o1_h, o2_h, o3_h)
        for j, (_, _, _, _, e, m) in enumerate(jobs):
            _emit_segsum(core, tid, lane, tabs[j], srcs[j], dsts[j],
                         inits[j], outs[j], win_src, win_dst, comp_src,
                         comp_dst, rows_v0, rows_v1, accum, sem0, sem1, e, m)

    run = pl.kernel(
        body,
        out_type=[jax.ShapeDtypeStruct((n0, h), jnp.float32),
                  jax.ShapeDtypeStruct((n1, h), jnp.float32),
                  jax.ShapeDtypeStruct((n1, h), jnp.float32),
                  jax.ShapeDtypeStruct((n2, h), jnp.float32)],
        mesh=mesh,
        compiler_params=pltpu.CompilerParams(
            needs_layout_passes=False, use_tc_tiling_on_sc=False),
        scratch_types=[
            pltpu.VMEM((_W,), jnp.int32),
            pltpu.VMEM((_W,), jnp.int32),
            pltpu.VMEM((cap,), jnp.int32),
            pltpu.VMEM((capg, _GRP), jnp.int32),
            pltpu.VMEM((_GRP, h), jnp.float32),
            pltpu.VMEM((_GRP, h), jnp.float32),
            pltpu.VMEM_SHARED((_CMAX + 8, h), jnp.float32),
            pltpu.SemaphoreType.DMA,
            pltpu.SemaphoreType.DMA,
        ],
    )
    return run(yu0, up0s, up0d, yu1, up1s, up1d,
               yb1s, b1s, b1d, yb1self, yb2s, b2s, b2d, yb2self)


def kernel(x0, x1, x2, up0, up1, b1_src, b1_dst, b2_src, b2_dst,
           batch0, batch1, batch2, params, lin1):
    ns = (x0.shape[0], x1.shape[0], x2.shape[0])
    srcs = [x0, x1, x2]
    sss = [None, None, None]
    for l in range(N_LAYERS):
        pl0, pl1, pl2 = params[l][0], params[l][1], params[l][2]
        yu0, yb0, yb1s = _proj(
            srcs[0], [pl0["up"]["W1"], pl0["bd"]["W1"], pl1["bd"]["W1"]], sss[0])
        yu1, yb1, yb2s = _proj(
            srcs[1], [pl1["up"]["W1"], pl1["bd"]["W1"], pl2["bd"]["W1"]], sss[1])
        yu2, yb2 = _proj(
            srcs[2], [pl2["up"]["W1"], pl2["bd"]["W1"]], sss[2])
        a_up0, a_up1, a_bd1, a_bd2 = _layer_segsums_sc(
            yu0, up0[0], up0[1], ns[0], yu1, up1[0], up1[1], ns[1],
            yb1s, b1_src, b1_dst, yb1, yb2s, b2_src, b2_dst, yb2, ns[2])
        new_srcs, new_sss = [], []
        for d, (a_up, a_bd) in enumerate(((a_up0, yb0), (a_up1, a_bd1),
                                          (yu2, a_bd2))):
            z_u, ss_u = _p2(a_up, params[l][d]["up"])
            z_b, ss_b = _p2(a_bd, params[l][d]["bd"])
            c, ss3 = _p3(z_u, ss_u, z_b, ss_b, params[l][d]["comb"])
            new_srcs.append(c)
            new_sss.append(ss3)
        srcs, sss = new_srcs, new_sss
    pooled = [_p4(srcs[d], sss[d], b)
              for d, b in enumerate((batch0, batch1, batch2))]
    return _p5(pooled, lin1)


# speedup vs baseline: 2.7175x; 1.1051x over previous
import functools

import jax
import jax.numpy as jnp
from jax import lax
from jax.experimental import pallas as pl
from jax.experimental.pallas import tpu as pltpu
from jax.experimental.pallas import tpu_sc as plsc

HIDDEN = 64
IN_DIM = 128
N_LAYERS = 2
MAX_DIM = 2
B = 128
FHM = 2
EPS = 1e-5
BLK = 5000


def _scale_shift(s, ss, n, g, be):
    m = s / n
    v = ss / n - m * m
    sc = g * lax.rsqrt(v + EPS)
    sh = be - m * sc
    return jnp.stack([sc, sh], axis=0)


def _proj_body(nouts, nb, *refs):
    if len(refs) == 3 + nouts:
        x_ref, ss_ref, w_ref = refs[:3]
        xb = x_ref[...] * ss_ref[0:1, :] + ss_ref[1:2, :]
        xb = jnp.maximum(xb, 0.0)
    else:
        x_ref, w_ref = refs[:2]
        xb = x_ref[...]
    outs = refs[-nouts:]
    y = jnp.dot(xb, w_ref[...], preferred_element_type=jnp.float32)
    for k, o_ref in enumerate(outs):
        o_ref[...] = y[:, k * HIDDEN:(k + 1) * HIDDEN]


def _proj(x, ws, ss=None):
    n, k = x.shape
    nb = n // BLK
    wcat = jnp.concatenate(ws, axis=1)
    nouts = len(ws)
    in_specs = [pl.BlockSpec((BLK, k), lambda i: (i, 0))]
    args = [x]
    if ss is not None:
        in_specs.append(pl.BlockSpec((2, k), lambda i: (0, 0)))
        args.append(ss)
    in_specs.append(pl.BlockSpec((k, nouts * HIDDEN), lambda i: (0, 0)))
    args.append(wcat)
    return pl.pallas_call(
        functools.partial(_proj_body, nouts, nb),
        grid=(nb,),
        in_specs=in_specs,
        out_specs=[pl.BlockSpec((BLK, HIDDEN), lambda i: (i, 0))] * nouts,
        out_shape=[jax.ShapeDtypeStruct((n, HIDDEN), jnp.float32)] * nouts,
    )(*args)


def _p23_body(n, nb, au_ref, ab_ref, gu_ref, wu_ref, gb_ref, wb_ref,
              wc_ref, g3_ref, c_ref, ss3_ref,
              acc_au, acc_ab, acc_zu, acc_zb, acc_c, ssv):
    s = pl.program_id(0)
    i = pl.program_id(1)
    au = au_ref[...]
    ab = ab_ref[...]
    fn = float(n)

    @pl.when((s == 0) & (i == 0))
    def _():
        acc_au[...] = jnp.zeros_like(acc_au)
        acc_ab[...] = jnp.zeros_like(acc_ab)

    @pl.when(s == 0)
    def _():
        acc_au[...] += jnp.stack(
            [jnp.sum(au, axis=0), jnp.sum(au * au, axis=0)], axis=0)
        acc_ab[...] += jnp.stack(
            [jnp.sum(ab, axis=0), jnp.sum(ab * ab, axis=0)], axis=0)
        c_ref[...] = jnp.zeros_like(c_ref)
        ss3_ref[...] = jnp.zeros_like(ss3_ref)

    @pl.when((s == 1) & (i == 0))
    def _():
        su, sb = acc_au[...], acc_ab[...]
        ssv[0:2, :] = _scale_shift(su[0], su[1], fn, gu_ref[0], gu_ref[1])
        ssv[2:4, :] = _scale_shift(sb[0], sb[1], fn, gb_ref[0], gb_ref[1])
        acc_zu[...] = jnp.zeros_like(acc_zu)
        acc_zb[...] = jnp.zeros_like(acc_zb)

    def _z(a, sl, w_ref):
        h1 = jnp.maximum(a * ssv[sl:sl + 1, :] + ssv[sl + 1:sl + 2, :], 0.0)
        return jnp.dot(h1, w_ref[...], preferred_element_type=jnp.float32)

    @pl.when(s == 1)
    def _():
        zu = _z(au, 0, wu_ref)
        zb = _z(ab, 2, wb_ref)
        acc_zu[...] += jnp.stack(
            [jnp.sum(zu, axis=0), jnp.sum(zu * zu, axis=0)], axis=0)
        acc_zb[...] += jnp.stack(
            [jnp.sum(zb, axis=0), jnp.sum(zb * zb, axis=0)], axis=0)

    @pl.when((s == 2) & (i == 0))
    def _():
        su, sb = acc_zu[...], acc_zb[...]
        ssv[4:6, :] = _scale_shift(su[0], su[1], fn, gu_ref[2], gu_ref[3])
        ssv[6:8, :] = _scale_shift(sb[0], sb[1], fn, gb_ref[2], gb_ref[3])
        acc_c[...] = jnp.zeros_like(acc_c)

    @pl.when(s == 2)
    def _():
        zu = _z(au, 0, wu_ref)
        zb = _z(ab, 2, wb_ref)
        h2u = jnp.maximum(zu * ssv[4:5, :] + ssv[5:6, :], 0.0)
        h2b = jnp.maximum(zb * ssv[6:7, :] + ssv[7:8, :], 0.0)
        wc = wc_ref[...]
        c = (jnp.dot(h2u, wc[:HIDDEN], preferred_element_type=jnp.float32)
             + jnp.dot(h2b, wc[HIDDEN:], preferred_element_type=jnp.float32))
        c_ref[...] = c
        acc_c[...] += jnp.stack(
            [jnp.sum(c, axis=0), jnp.sum(c * c, axis=0)], axis=0)
        st = acc_c[...]
        ss3_ref[...] = _scale_shift(st[0], st[1], fn, g3_ref[0], g3_ref[1])


def _p23(a_up, p_up, a_bd, p_bd, pc):
    n = a_up.shape[0]
    nb = n // BLK
    gu = jnp.stack([p_up["g1"], p_up["be1"], p_up["g2"], p_up["be2"]], axis=0)
    gb = jnp.stack([p_bd["g1"], p_bd["be1"], p_bd["g2"], p_bd["be2"]], axis=0)
    g3 = jnp.stack([pc["g"], pc["be"]], axis=0)
    return pl.pallas_call(
        functools.partial(_p23_body, n, nb),
        grid=(3, nb),
        in_specs=[
            pl.BlockSpec((BLK, HIDDEN), lambda s, i: (i, 0)),
            pl.BlockSpec((BLK, HIDDEN), lambda s, i: (i, 0)),
            pl.BlockSpec((4, HIDDEN), lambda s, i: (0, 0)),
            pl.BlockSpec((HIDDEN, HIDDEN), lambda s, i: (0, 0)),
            pl.BlockSpec((4, HIDDEN), lambda s, i: (0, 0)),
            pl.BlockSpec((HIDDEN, HIDDEN), lambda s, i: (0, 0)),
            pl.BlockSpec((2 * HIDDEN, HIDDEN), lambda s, i: (0, 0)),
            pl.BlockSpec((2, HIDDEN), lambda s, i: (0, 0)),
        ],
        out_specs=[
            pl.BlockSpec((BLK, HIDDEN), lambda s, i: (i, 0)),
            pl.BlockSpec((2, HIDDEN), lambda s, i: (0, 0)),
        ],
        out_shape=[
            jax.ShapeDtypeStruct((n, HIDDEN), jnp.float32),
            jax.ShapeDtypeStruct((2, HIDDEN), jnp.float32),
        ],
        scratch_shapes=[pltpu.VMEM((2, HIDDEN), jnp.float32)] * 5
        + [pltpu.VMEM((8, HIDDEN), jnp.float32)],
    )(a_up, a_bd, gu, p_up["W2"], gb, p_bd["W2"], pc["W"], g3)


def _p4_body(nb, c_ref, ss_ref, ids_ref, out_ref, acc):
    i = pl.program_id(0)

    @pl.when(i == 0)
    def _():
        acc[...] = jnp.zeros_like(acc)

    cb = jnp.maximum(c_ref[...] * ss_ref[0:1, :] + ss_ref[1:2, :], 0.0)
    ids = ids_ref[0, 0, :]
    onehot_t = (lax.broadcasted_iota(jnp.int32, (B, BLK), 0)
                == ids[None, :]).astype(jnp.float32)
    acc[...] += jnp.dot(onehot_t, cb, preferred_element_type=jnp.float32)
    out_ref[...] = acc[...]


def _p4(c, ss, batch):
    n = c.shape[0]
    nb = n // BLK
    ids3 = batch.reshape(nb, 1, BLK)
    return pl.pallas_call(
        functools.partial(_p4_body, nb),
        grid=(nb,),
        in_specs=[
            pl.BlockSpec((BLK, HIDDEN), lambda i: (i, 0)),
            pl.BlockSpec((2, HIDDEN), lambda i: (0, 0)),
            pl.BlockSpec((1, 1, BLK), lambda i: (i, 0, 0)),
        ],
        out_specs=pl.BlockSpec((B, HIDDEN), lambda i: (0, 0)),
        out_shape=jax.ShapeDtypeStruct((B, HIDDEN), jnp.float32),
        scratch_shapes=[pltpu.VMEM((B, HIDDEN), jnp.float32)],
    )(c, ss, ids3)


def _p5_body(p0, p1, p2, w0, w1, w2, b0, b1, b2, out_ref):
    acc = jnp.zeros((B, FHM * HIDDEN), jnp.float32)
    for p, w, b in ((p0, w0, b0), (p1, w1, b1), (p2, w2, b2)):
        acc += jnp.maximum(
            jnp.dot(p[...], w[...], preferred_element_type=jnp.float32)
            + b[...], 0.0)
    out_ref[...] = acc


def _p5(pooled, lin1):
    args = list(pooled) + [lin1[d]["W"] for d in range(3)] \
        + [lin1[d]["b"].reshape(1, -1) for d in range(3)]
    return pl.pallas_call(
        _p5_body,
        out_shape=jax.ShapeDtypeStruct((B, FHM * HIDDEN), jnp.float32),
    )(*args)


_NTILES = 16
_GRP = 128
_W = 2000
_CMAX = 12480


def _seg_cfg(m):
    nch = -(-m // _CMAX)
    nch += nch % 2
    nch = max(2, nch)
    c = ((-(-m // nch)) + 15) // 16 * 16
    tail = m - (nch - 1) * c
    assert 0 < tail <= c and tail % 16 == 0 and c <= _CMAX
    return nch, c, tail


_UNROLL = 5


def _emit_segsum(core, tid, lane, table_h, src_h, dst_h, init_h, out_h,
                 win_src, win_dst, comp_src, comp_dst, rows_v0, rows_v1,
                 accum, sem0, sem1, e, m):
    nch, c, tail = _seg_cfg(m)
    npass = nch // 2
    epw = e // _NTILES
    nwin = epw // _W
    assert epw % _W == 0
    rpt, rpt_t = c // _NTILES, tail // _NTILES
    estart = tid * epw
    trash_src = lane * 8
    trash_dst = _CMAX + (lane & 7)

    for p in range(npass):
        chunk = 2 * p + core
        lo = chunk * c
        last = p == npass - 1

        if last:
            @pl.when(core == 0)
            def _():
                pltpu.sync_copy(init_h.at[pl.ds(lo + tid * rpt, rpt)],
                                accum.at[pl.ds(tid * rpt, rpt)])

            @pl.when(core == 1)
            def _():
                pltpu.sync_copy(init_h.at[pl.ds(lo + tid * rpt_t, rpt_t)],
                                accum.at[pl.ds(tid * rpt_t, rpt_t)])
        else:
            pltpu.sync_copy(init_h.at[pl.ds(lo + tid * rpt, rpt)],
                            accum.at[pl.ds(tid * rpt, rpt)])
        plsc.subcore_barrier()

        def win_body(wi, cur):
            base = estart + wi * _W
            pltpu.sync_copy(src_h.at[pl.ds(base, _W)], win_src)
            pltpu.sync_copy(dst_h.at[pl.ds(base, _W)], win_dst)

            def vbody(v, cur):
                off0 = v * (16 * _UNROLL)
                ms, cums, cnts, srcs, locs = [], [], [], [], []
                for k in range(_UNROLL):
                    off = off0 + k * 16
                    dstv = win_dst[pl.ds(off, 16)]
                    srcv = win_src[pl.ds(off, 16)]
                    local = dstv - lo
                    mask = (local >= 0) & (local < c)
                    mi = mask.astype(jnp.int32)
                    ms.append(mask)
                    cums.append(plsc.cumsum(mi))
                    cnts.append(jnp.sum(mi))
                    srcs.append(srcv)
                    locs.append(local)
                base_k = cur
                for k in range(_UNROLL):
                    pos = base_k + cums[k] - 1
                    plsc.store_scatter(comp_src, [pos], srcs[k], mask=ms[k])
                    plsc.store_scatter(
                        comp_dst, [pos >> 7, pos & 127], locs[k], mask=ms[k])
                    base_k = base_k + cnts[k]
                return base_k

            return lax.fori_loop(0, _W // (16 * _UNROLL), vbody, cur)

        cursor = lax.fori_loop(0, nwin, win_body, jnp.int32(0))

        for k in range(_GRP // 16):
            posk = cursor + k * 16 + lane
            full = posk >= 0
            plsc.store_scatter(comp_src, [posk], trash_src, mask=full)
            plsc.store_scatter(
                comp_dst, [posk >> 7, posk & 127], trash_dst, mask=full)
        ngroups = (cursor + (_GRP - 1)) >> 7

        def _issue(g, rows_v, sem):
            pltpu.async_copy(
                table_h.at[comp_src.at[pl.ds(g * _GRP, _GRP)]], rows_v, sem)

        def _wait(rows_v, sem):
            pltpu.make_async_copy(
                table_h.at[comp_src.at[pl.ds(0, _GRP)]], rows_v, sem).wait()

        @pl.when(ngroups > 0)
        def _():
            _issue(0, rows_v0, sem0)

        def gbody(k2, carry):
            g0 = k2 * 2

            @pl.when(g0 < ngroups)
            def _():
                _wait(rows_v0, sem0)

                @pl.when(g0 + 1 < ngroups)
                def _():
                    _issue(g0 + 1, rows_v1, sem1)
                pltpu.sync_copy(rows_v0, accum.at[comp_dst.at[g0]], add=True)

            @pl.when(g0 + 1 < ngroups)
            def _():
                _wait(rows_v1, sem1)

                @pl.when(g0 + 2 < ngroups)
                def _():
                    _issue(g0 + 2, rows_v0, sem0)
                pltpu.sync_copy(rows_v1, accum.at[comp_dst.at[g0 + 1]],
                                add=True)
            return carry

        lax.fori_loop(0, (ngroups + 1) >> 1, gbody, jnp.int32(0))
        plsc.subcore_barrier()

        if last:
            @pl.when(core == 0)
            def _():
                pltpu.sync_copy(accum.at[pl.ds(tid * rpt, rpt)],
                                out_h.at[pl.ds(lo + tid * rpt, rpt)])

            @pl.when(core == 1)
            def _():
                pltpu.sync_copy(accum.at[pl.ds(tid * rpt_t, rpt_t)],
                                out_h.at[pl.ds(lo + tid * rpt_t, rpt_t)])
        else:
            pltpu.sync_copy(accum.at[pl.ds(tid * rpt, rpt)],
                            out_h.at[pl.ds(lo + tid * rpt, rpt)])
        plsc.subcore_barrier()


def _layer_segsums_sc(yu0, up0s, up0d, n0, yu1, up1s, up1d, n1,
                      yb1s, b1s, b1d, yb1self, yb2s, b2s, b2d, yb2self, n2):
    h = HIDDEN
    jobs = [
        (yu0, up0s, up0d, yu0, up0s.shape[0], n0),
        (yu1, up1s, up1d, yu1, up1s.shape[0], n1),
        (yb1s, b1s, b1d, yb1self, b1s.shape[0], n1),
        (yb2s, b2s, b2d, yb2self, b2s.shape[0], n2),
    ]
    emax = max(j[4] for j in jobs)
    capg = emax // _NTILES // _GRP + 2
    cap = capg * _GRP
    mesh = plsc.VectorSubcoreMesh(core_axis_name="c", subcore_axis_name="s")

    def body(yu0_h, up0s_h, up0d_h, yu1_h, up1s_h, up1d_h,
             yb1s_h, b1s_h, b1d_h, yb1i_h, yb2s_h, b2s_h, b2d_h, yb2i_h,
             o0_h, o1_h, o2_h, o3_h,
             win_src, win_dst, comp_src, comp_dst, rows_v0, rows_v1,
             accum, sem0, sem1):
        core = lax.axis_index("c")
        tid = lax.axis_index("s")
        lane = lax.iota(jnp.int32, 16)
        tabs = (yu0_h, yu1_h, yb1s_h, yb2s_h)
        srcs = (up0s_h, up1s_h, b1s_h, b2s_h)
        dsts = (up0d_h, up1d_h, b1d_h, b2d_h)
        inits = (yu0_h, yu1_h, yb1i_h, yb2i_h)
        outs = (o0_h, o1_h, o2_h, o3_h)
        for j, (_, _, _, _, e, m) in enumerate(jobs):
            _emit_segsum(core, tid, lane, tabs[j], srcs[j], dsts[j],
                         inits[j], outs[j], win_src, win_dst, comp_src,
                         comp_dst, rows_v0, rows_v1, accum, sem0, sem1, e, m)

    run = pl.kernel(
        body,
        out_type=[jax.ShapeDtypeStruct((n0, h), jnp.float32),
                  jax.ShapeDtypeStruct((n1, h), jnp.float32),
                  jax.ShapeDtypeStruct((n1, h), jnp.float32),
                  jax.ShapeDtypeStruct((n2, h), jnp.float32)],
        mesh=mesh,
        compiler_params=pltpu.CompilerParams(
            needs_layout_passes=False, use_tc_tiling_on_sc=False),
        scratch_types=[
            pltpu.VMEM((_W,), jnp.int32),
            pltpu.VMEM((_W,), jnp.int32),
            pltpu.VMEM((cap,), jnp.int32),
            pltpu.VMEM((capg, _GRP), jnp.int32),
            pltpu.VMEM((_GRP, h), jnp.float32),
            pltpu.VMEM((_GRP, h), jnp.float32),
            pltpu.VMEM_SHARED((_CMAX + 8, h), jnp.float32),
            pltpu.SemaphoreType.DMA,
            pltpu.SemaphoreType.DMA,
        ],
    )
    return run(yu0, up0s, up0d, yu1, up1s, up1d,
               yb1s, b1s, b1d, yb1self, yb2s, b2s, b2d, yb2self)


def kernel(x0, x1, x2, up0, up1, b1_src, b1_dst, b2_src, b2_dst,
           batch0, batch1, batch2, params, lin1):
    ns = (x0.shape[0], x1.shape[0], x2.shape[0])
    srcs = [x0, x1, x2]
    sss = [None, None, None]
    for l in range(N_LAYERS):
        pl0, pl1, pl2 = params[l][0], params[l][1], params[l][2]
        yu0, yb0, yb1s = _proj(
            srcs[0], [pl0["up"]["W1"], pl0["bd"]["W1"], pl1["bd"]["W1"]], sss[0])
        yu1, yb1, yb2s = _proj(
            srcs[1], [pl1["up"]["W1"], pl1["bd"]["W1"], pl2["bd"]["W1"]], sss[1])
        yu2, yb2 = _proj(
            srcs[2], [pl2["up"]["W1"], pl2["bd"]["W1"]], sss[2])
        a_up0, a_up1, a_bd1, a_bd2 = _layer_segsums_sc(
            yu0, up0[0], up0[1], ns[0], yu1, up1[0], up1[1], ns[1],
            yb1s, b1_src, b1_dst, yb1, yb2s, b2_src, b2_dst, yb2, ns[2])
        new_srcs, new_sss = [], []
        for d, (a_up, a_bd) in enumerate(((a_up0, yb0), (a_up1, a_bd1),
                                          (yu2, a_bd2))):
            c, ss3 = _p23(a_up, params[l][d]["up"], a_bd,
                          params[l][d]["bd"], params[l][d]["comb"])
            new_srcs.append(c)
            new_sss.append(ss3)
        srcs, sss = new_srcs, new_sss
    pooled = [_p4(srcs[d], sss[d], b)
              for d, b in enumerate((batch0, batch1, batch2))]
    return _p5(pooled, lin1)


# speedup vs baseline: 2.8243x; 1.0393x over previous
import functools

import jax
import jax.numpy as jnp
from jax import lax
from jax.experimental import pallas as pl
from jax.experimental.pallas import tpu as pltpu
from jax.experimental.pallas import tpu_sc as plsc

HIDDEN = 64
IN_DIM = 128
N_LAYERS = 2
MAX_DIM = 2
B = 128
FHM = 2
EPS = 1e-5
BLK = 5000


def _scale_shift(s, ss, n, g, be):
    m = s / n
    v = ss / n - m * m
    sc = g * lax.rsqrt(v + EPS)
    sh = be - m * sc
    return jnp.stack([sc, sh], axis=0)


def _proj_body(nouts, nb, *refs):
    if len(refs) == 3 + nouts:
        x_ref, ss_ref, w_ref = refs[:3]
        xb = x_ref[...] * ss_ref[0:1, :] + ss_ref[1:2, :]
        xb = jnp.maximum(xb, 0.0)
    else:
        x_ref, w_ref = refs[:2]
        xb = x_ref[...]
    outs = refs[-nouts:]
    y = jnp.dot(xb, w_ref[...], preferred_element_type=jnp.float32)
    for k, o_ref in enumerate(outs):
        o_ref[...] = y[:, k * HIDDEN:(k + 1) * HIDDEN]


def _proj(x, ws, ss=None):
    n, k = x.shape
    nb = n // BLK
    wcat = jnp.concatenate(ws, axis=1)
    nouts = len(ws)
    in_specs = [pl.BlockSpec((BLK, k), lambda i: (i, 0))]
    args = [x]
    if ss is not None:
        in_specs.append(pl.BlockSpec((2, k), lambda i: (0, 0)))
        args.append(ss)
    in_specs.append(pl.BlockSpec((k, nouts * HIDDEN), lambda i: (0, 0)))
    args.append(wcat)
    return pl.pallas_call(
        functools.partial(_proj_body, nouts, nb),
        grid=(nb,),
        in_specs=in_specs,
        out_specs=[pl.BlockSpec((BLK, HIDDEN), lambda i: (i, 0))] * nouts,
        out_shape=[jax.ShapeDtypeStruct((n, HIDDEN), jnp.float32)] * nouts,
    )(*args)


def _p23_body(n, nb, au_ref, ab_ref, g12_ref, w2_ref, wc_ref, g3_ref,
              c_ref, ss3_ref, acc_a, acc_z, acc_c, ssv):
    s = pl.program_id(0)
    i = pl.program_id(1)
    acat = jnp.concatenate([au_ref[...], ab_ref[...]], axis=1)
    fn = float(n)

    @pl.when((s == 0) & (i == 0))
    def _():
        acc_a[...] = jnp.zeros_like(acc_a)

    @pl.when(s == 0)
    def _():
        acc_a[...] += jnp.stack(
            [jnp.sum(acat, axis=0), jnp.sum(acat * acat, axis=0)], axis=0)
        c_ref[...] = jnp.zeros_like(c_ref)
        ss3_ref[...] = jnp.zeros_like(ss3_ref)

    @pl.when((s == 1) & (i == 0))
    def _():
        st = acc_a[...]
        ssv[0:2, :] = _scale_shift(st[0], st[1], fn, g12_ref[0], g12_ref[1])
        acc_z[...] = jnp.zeros_like(acc_z)

    def _zcat():
        h1 = jnp.maximum(acat * ssv[0:1, :] + ssv[1:2, :], 0.0)
        return jnp.dot(h1, w2_ref[...], preferred_element_type=jnp.float32)

    @pl.when(s == 1)
    def _():
        z = _zcat()
        acc_z[...] += jnp.stack(
            [jnp.sum(z, axis=0), jnp.sum(z * z, axis=0)], axis=0)

    @pl.when((s == 2) & (i == 0))
    def _():
        st = acc_z[...]
        ssv[2:4, :] = _scale_shift(st[0], st[1], fn, g12_ref[2], g12_ref[3])
        acc_c[...] = jnp.zeros_like(acc_c)

    @pl.when(s == 2)
    def _():
        z = _zcat()
        h2 = jnp.maximum(z * ssv[2:3, :] + ssv[3:4, :], 0.0)
        c = jnp.dot(h2, wc_ref[...], preferred_element_type=jnp.float32)
        c_ref[...] = c
        acc_c[...] += jnp.stack(
            [jnp.sum(c, axis=0), jnp.sum(c * c, axis=0)], axis=0)
        st = acc_c[...]
        ss3_ref[...] = _scale_shift(st[0], st[1], fn, g3_ref[0], g3_ref[1])


def _p23(a_up, p_up, a_bd, p_bd, pc):
    n = a_up.shape[0]
    nb = n // BLK
    h = HIDDEN
    g12 = jnp.stack([
        jnp.concatenate([p_up["g1"], p_bd["g1"]]),
        jnp.concatenate([p_up["be1"], p_bd["be1"]]),
        jnp.concatenate([p_up["g2"], p_bd["g2"]]),
        jnp.concatenate([p_up["be2"], p_bd["be2"]])], axis=0)
    w2 = jnp.zeros((2 * h, 2 * h), jnp.float32)
    w2 = w2.at[:h, :h].set(p_up["W2"]).at[h:, h:].set(p_bd["W2"])
    g3 = jnp.stack([pc["g"], pc["be"]], axis=0)
    return pl.pallas_call(
        functools.partial(_p23_body, n, nb),
        grid=(3, nb),
        in_specs=[
            pl.BlockSpec((BLK, h), lambda s, i: (i, 0)),
            pl.BlockSpec((BLK, h), lambda s, i: (i, 0)),
            pl.BlockSpec((4, 2 * h), lambda s, i: (0, 0)),
            pl.BlockSpec((2 * h, 2 * h), lambda s, i: (0, 0)),
            pl.BlockSpec((2 * h, h), lambda s, i: (0, 0)),
            pl.BlockSpec((2, h), lambda s, i: (0, 0)),
        ],
        out_specs=[
            pl.BlockSpec((BLK, h), lambda s, i: (i, 0)),
            pl.BlockSpec((2, h), lambda s, i: (0, 0)),
        ],
        out_shape=[
            jax.ShapeDtypeStruct((n, h), jnp.float32),
            jax.ShapeDtypeStruct((2, h), jnp.float32),
        ],
        scratch_shapes=[
            pltpu.VMEM((2, 2 * h), jnp.float32),
            pltpu.VMEM((2, 2 * h), jnp.float32),
            pltpu.VMEM((2, h), jnp.float32),
            pltpu.VMEM((4, 2 * h), jnp.float32),
        ],
    )(a_up, a_bd, g12, w2, pc["W"], g3)


def _p4_body(nb, c_ref, ss_ref, ids_ref, out_ref, acc):
    i = pl.program_id(0)

    @pl.when(i == 0)
    def _():
        acc[...] = jnp.zeros_like(acc)

    cb = jnp.maximum(c_ref[...] * ss_ref[0:1, :] + ss_ref[1:2, :], 0.0)
    ids = ids_ref[0, 0, :]
    onehot_t = (lax.broadcasted_iota(jnp.int32, (B, BLK), 0)
                == ids[None, :]).astype(jnp.float32)
    acc[...] += jnp.dot(onehot_t, cb, preferred_element_type=jnp.float32)
    out_ref[...] = acc[...]


def _p4(c, ss, batch):
    n = c.shape[0]
    nb = n // BLK
    ids3 = batch.reshape(nb, 1, BLK)
    return pl.pallas_call(
        functools.partial(_p4_body, nb),
        grid=(nb,),
        in_specs=[
            pl.BlockSpec((BLK, HIDDEN), lambda i: (i, 0)),
            pl.BlockSpec((2, HIDDEN), lambda i: (0, 0)),
            pl.BlockSpec((1, 1, BLK), lambda i: (i, 0, 0)),
        ],
        out_specs=pl.BlockSpec((B, HIDDEN), lambda i: (0, 0)),
        out_shape=jax.ShapeDtypeStruct((B, HIDDEN), jnp.float32),
        scratch_shapes=[pltpu.VMEM((B, HIDDEN), jnp.float32)],
    )(c, ss, ids3)


def _p5_body(p0, p1, p2, w0, w1, w2, b0, b1, b2, out_ref):
    acc = jnp.zeros((B, FHM * HIDDEN), jnp.float32)
    for p, w, b in ((p0, w0, b0), (p1, w1, b1), (p2, w2, b2)):
        acc += jnp.maximum(
            jnp.dot(p[...], w[...], preferred_element_type=jnp.float32)
            + b[...], 0.0)
    out_ref[...] = acc


def _p5(pooled, lin1):
    args = list(pooled) + [lin1[d]["W"] for d in range(3)] \
        + [lin1[d]["b"].reshape(1, -1) for d in range(3)]
    return pl.pallas_call(
        _p5_body,
        out_shape=jax.ShapeDtypeStruct((B, FHM * HIDDEN), jnp.float32),
    )(*args)


_NTILES = 16
_GRP = 128
_W = 2000
_CMAX = 12480


def _seg_cfg(m):
    nch = -(-m // _CMAX)
    nch += nch % 2
    nch = max(2, nch)
    c = ((-(-m // nch)) + 15) // 16 * 16
    tail = m - (nch - 1) * c
    assert 0 < tail <= c and tail % 16 == 0 and c <= _CMAX
    return nch, c, tail


_UNROLL = 5


def _emit_segsum(core, tid, lane, table_h, src_h, dst_h, init_h, out_h,
                 win_src, win_dst, comp_src, comp_dst, rows_v0, rows_v1,
                 accum, sem0, sem1, e, m):
    nch, c, tail = _seg_cfg(m)
    npass = nch // 2
    epw = e // _NTILES
    nwin = epw // _W
    assert epw % _W == 0
    rpt, rpt_t = c // _NTILES, tail // _NTILES
    estart = tid * epw
    trash_src = lane * 8
    trash_dst = _CMAX + (lane & 7)

    for p in range(npass):
        chunk = 2 * p + core
        lo = chunk * c
        last = p == npass - 1

        if last:
            @pl.when(core == 0)
            def _():
                pltpu.sync_copy(init_h.at[pl.ds(lo + tid * rpt, rpt)],
                                accum.at[pl.ds(tid * rpt, rpt)])

            @pl.when(core == 1)
            def _():
                pltpu.sync_copy(init_h.at[pl.ds(lo + tid * rpt_t, rpt_t)],
                                accum.at[pl.ds(tid * rpt_t, rpt_t)])
        else:
            pltpu.sync_copy(init_h.at[pl.ds(lo + tid * rpt, rpt)],
                            accum.at[pl.ds(tid * rpt, rpt)])
        plsc.subcore_barrier()

        def win_body(wi, cur):
            base = estart + wi * _W
            pltpu.sync_copy(src_h.at[pl.ds(base, _W)], win_src)
            pltpu.sync_copy(dst_h.at[pl.ds(base, _W)], win_dst)

            def vbody(v, cur):
                off0 = v * (16 * _UNROLL)
                ms, cums, cnts, srcs, locs = [], [], [], [], []
                for k in range(_UNROLL):
                    off = off0 + k * 16
                    dstv = win_dst[pl.ds(off, 16)]
                    srcv = win_src[pl.ds(off, 16)]
                    local = dstv - lo
                    mask = (local >= 0) & (local < c)
                    mi = mask.astype(jnp.int32)
                    ms.append(mask)
                    cums.append(plsc.cumsum(mi))
                    cnts.append(jnp.sum(mi))
                    srcs.append(srcv)
                    locs.append(local)
                base_k = cur
                for k in range(_UNROLL):
                    pos = base_k + cums[k] - 1
                    plsc.store_scatter(comp_src, [pos], srcs[k], mask=ms[k])
                    plsc.store_scatter(
                        comp_dst, [pos >> 7, pos & 127], locs[k], mask=ms[k])
                    base_k = base_k + cnts[k]
                return base_k

            return lax.fori_loop(0, _W // (16 * _UNROLL), vbody, cur)

        cursor = lax.fori_loop(0, nwin, win_body, jnp.int32(0))

        for k in range(_GRP // 16):
            posk = cursor + k * 16 + lane
            full = posk >= 0
            plsc.store_scatter(comp_src, [posk], trash_src, mask=full)
            plsc.store_scatter(
                comp_dst, [posk >> 7, posk & 127], trash_dst, mask=full)
        ngroups = (cursor + (_GRP - 1)) >> 7

        def _issue(g, rows_v, sem):
            pltpu.async_copy(
                table_h.at[comp_src.at[pl.ds(g * _GRP, _GRP)]], rows_v, sem)

        def _wait(rows_v, sem):
            pltpu.make_async_copy(
                table_h.at[comp_src.at[pl.ds(0, _GRP)]], rows_v, sem).wait()

        @pl.when(ngroups > 0)
        def _():
            _issue(0, rows_v0, sem0)

        def gbody(k2, carry):
            g0 = k2 * 2

            @pl.when(g0 < ngroups)
            def _():
                _wait(rows_v0, sem0)

                @pl.when(g0 + 1 < ngroups)
                def _():
                    _issue(g0 + 1, rows_v1, sem1)
                pltpu.sync_copy(rows_v0, accum.at[comp_dst.at[g0]], add=True)

            @pl.when(g0 + 1 < ngroups)
            def _():
                _wait(rows_v1, sem1)

                @pl.when(g0 + 2 < ngroups)
                def _():
                    _issue(g0 + 2, rows_v0, sem0)
                pltpu.sync_copy(rows_v1, accum.at[comp_dst.at[g0 + 1]],
                                add=True)
            return carry

        lax.fori_loop(0, (ngroups + 1) >> 1, gbody, jnp.int32(0))
        plsc.subcore_barrier()

        if last:
            @pl.when(core == 0)
            def _():
                pltpu.sync_copy(accum.at[pl.ds(tid * rpt, rpt)],
                                out_h.at[pl.ds(lo + tid * rpt, rpt)])

            @pl.when(core == 1)
            def _():
                pltpu.sync_copy(accum.at[pl.ds(tid * rpt_t, rpt_t)],
                                out_h.at[pl.ds(lo + tid * rpt_t, rpt_t)])
        else:
            pltpu.sync_copy(accum.at[pl.ds(tid * rpt, rpt)],
                            out_h.at[pl.ds(lo + tid * rpt, rpt)])
        plsc.subcore_barrier()


def _layer_segsums_sc(yu0, up0s, up0d, n0, yu1, up1s, up1d, n1,
                      yb1s, b1s, b1d, yb1self, yb2s, b2s, b2d, yb2self, n2):
    h = HIDDEN
    jobs = [
        (yu0, up0s, up0d, yu0, up0s.shape[0], n0),
        (yu1, up1s, up1d, yu1, up1s.shape[0], n1),
        (yb1s, b1s, b1d, yb1self, b1s.shape[0], n1),
        (yb2s, b2s, b2d, yb2self, b2s.shape[0], n2),
    ]
    emax = max(j[4] for j in jobs)
    capg = emax // _NTILES // _GRP + 2
    cap = capg * _GRP
    mesh = plsc.VectorSubcoreMesh(core_axis_name="c", subcore_axis_name="s")

    def body(yu0_h, up0s_h, up0d_h, yu1_h, up1s_h, up1d_h,
             yb1s_h, b1s_h, b1d_h, yb1i_h, yb2s_h, b2s_h, b2d_h, yb2i_h,
             o0_h, o1_h, o2_h, o3_h,
             win_src, win_dst, comp_src, comp_dst, rows_v0, rows_v1,
             accum, sem0, sem1):
        core = lax.axis_index("c")
        tid = lax.axis_index("s")
        lane = lax.iota(jnp.int32, 16)
        tabs = (yu0_h, yu1_h, yb1s_h, yb2s_h)
        srcs = (up0s_h, up1s_h, b1s_h, b2s_h)
        dsts = (up0d_h, up1d_h, b1d_h, b2d_h)
        inits = (yu0_h, yu1_h, yb1i_h, yb2i_h)
        outs = (o0_h, o1_h, o2_h, o3_h)
        for j, (_, _, _, _, e, m) in enumerate(jobs):
            _emit_segsum(core, tid, lane, tabs[j], srcs[j], dsts[j],
                         inits[j], outs[j], win_src, win_dst, comp_src,
                         comp_dst, rows_v0, rows_v1, accum, sem0, sem1, e, m)

    run = pl.kernel(
        body,
        out_type=[jax.ShapeDtypeStruct((n0, h), jnp.float32),
                  jax.ShapeDtypeStruct((n1, h), jnp.float32),
                  jax.ShapeDtypeStruct((n1, h), jnp.float32),
                  jax.ShapeDtypeStruct((n2, h), jnp.float32)],
        mesh=mesh,
        compiler_params=pltpu.CompilerParams(
            needs_layout_passes=False, use_tc_tiling_on_sc=False),
        scratch_types=[
            pltpu.VMEM((_W,), jnp.int32),
            pltpu.VMEM((_W,), jnp.int32),
            pltpu.VMEM((cap,), jnp.int32),
            pltpu.VMEM((capg, _GRP), jnp.int32),
            pltpu.VMEM((_GRP, h), jnp.float32),
            pltpu.VMEM((_GRP, h), jnp.float32),
            pltpu.VMEM_SHARED((_CMAX + 8, h), jnp.float32),
            pltpu.SemaphoreType.DMA,
            pltpu.SemaphoreType.DMA,
        ],
    )
    return run(yu0, up0s, up0d, yu1, up1s, up1d,
               yb1s, b1s, b1d, yb1self, yb2s, b2s, b2d, yb2self)


def kernel(x0, x1, x2, up0, up1, b1_src, b1_dst, b2_src, b2_dst,
           batch0, batch1, batch2, params, lin1):
    ns = (x0.shape[0], x1.shape[0], x2.shape[0])
    srcs = [x0, x1, x2]
    sss = [None, None, None]
    for l in range(N_LAYERS):
        pl0, pl1, pl2 = params[l][0], params[l][1], params[l][2]
        yu0, yb0, yb1s = _proj(
            srcs[0], [pl0["up"]["W1"], pl0["bd"]["W1"], pl1["bd"]["W1"]], sss[0])
        yu1, yb1, yb2s = _proj(
            srcs[1], [pl1["up"]["W1"], pl1["bd"]["W1"], pl2["bd"]["W1"]], sss[1])
        yu2, yb2 = _proj(
            srcs[2], [pl2["up"]["W1"], pl2["bd"]["W1"]], sss[2])
        a_up0, a_up1, a_bd1, a_bd2 = _layer_segsums_sc(
            yu0, up0[0], up0[1], ns[0], yu1, up1[0], up1[1], ns[1],
            yb1s, b1_src, b1_dst, yb1, yb2s, b2_src, b2_dst, yb2, ns[2])
        new_srcs, new_sss = [], []
        for d, (a_up, a_bd) in enumerate(((a_up0, yb0), (a_up1, a_bd1),
                                          (yu2, a_bd2))):
            c, ss3 = _p23(a_up, params[l][d]["up"], a_bd,
                          params[l][d]["bd"], params[l][d]["comb"])
            new_srcs.append(c)
            new_sss.append(ss3)
        srcs, sss = new_srcs, new_sss
    pooled = [_p4(srcs[d], sss[d], b)
              for d, b in enumerate((batch0, batch1, batch2))]
    return _p5(pooled, lin1)


# speedup vs baseline: 2.8897x; 1.0232x over previous
import functools

import jax
import jax.numpy as jnp
from jax import lax
from jax.experimental import pallas as pl
from jax.experimental.pallas import tpu as pltpu
from jax.experimental.pallas import tpu_sc as plsc

HIDDEN = 64
IN_DIM = 128
N_LAYERS = 2
MAX_DIM = 2
B = 128
FHM = 2
EPS = 1e-5
BLK = 10000


def _scale_shift(s, ss, n, g, be):
    m = s / n
    v = ss / n - m * m
    sc = g * lax.rsqrt(v + EPS)
    sh = be - m * sc
    return jnp.stack([sc, sh], axis=0)


def _proj_body(nouts, nb, *refs):
    if len(refs) == 3 + nouts:
        x_ref, ss_ref, w_ref = refs[:3]
        xb = x_ref[...] * ss_ref[0:1, :] + ss_ref[1:2, :]
        xb = jnp.maximum(xb, 0.0)
    else:
        x_ref, w_ref = refs[:2]
        xb = x_ref[...]
    outs = refs[-nouts:]
    y = jnp.dot(xb, w_ref[...], preferred_element_type=jnp.float32)
    for k, o_ref in enumerate(outs):
        o_ref[...] = y[:, k * HIDDEN:(k + 1) * HIDDEN]


def _proj(x, ws, ss=None):
    n, k = x.shape
    nb = n // BLK
    wcat = jnp.concatenate(ws, axis=1)
    nouts = len(ws)
    in_specs = [pl.BlockSpec((BLK, k), lambda i: (i, 0))]
    args = [x]
    if ss is not None:
        in_specs.append(pl.BlockSpec((2, k), lambda i: (0, 0)))
        args.append(ss)
    in_specs.append(pl.BlockSpec((k, nouts * HIDDEN), lambda i: (0, 0)))
    args.append(wcat)
    return pl.pallas_call(
        functools.partial(_proj_body, nouts, nb),
        grid=(nb,),
        in_specs=in_specs,
        out_specs=[pl.BlockSpec((BLK, HIDDEN), lambda i: (i, 0))] * nouts,
        out_shape=[jax.ShapeDtypeStruct((n, HIDDEN), jnp.float32)] * nouts,
    )(*args)


def _p23_body(n, nb, au_ref, ab_ref, g12_ref, w2_ref, wc_ref, g3_ref,
              c_ref, ss3_ref, acc_a, acc_z, acc_c, ssv):
    s = pl.program_id(0)
    i = pl.program_id(1)
    acat = jnp.concatenate([au_ref[...], ab_ref[...]], axis=1)
    fn = float(n)

    @pl.when((s == 0) & (i == 0))
    def _():
        acc_a[...] = jnp.zeros_like(acc_a)

    @pl.when(s == 0)
    def _():
        acc_a[...] += jnp.stack(
            [jnp.sum(acat, axis=0), jnp.sum(acat * acat, axis=0)], axis=0)
        c_ref[...] = jnp.zeros_like(c_ref)
        ss3_ref[...] = jnp.zeros_like(ss3_ref)

    @pl.when((s == 1) & (i == 0))
    def _():
        st = acc_a[...]
        ssv[0:2, :] = _scale_shift(st[0], st[1], fn, g12_ref[0], g12_ref[1])
        acc_z[...] = jnp.zeros_like(acc_z)

    def _zcat():
        h1 = jnp.maximum(acat * ssv[0:1, :] + ssv[1:2, :], 0.0)
        return jnp.dot(h1, w2_ref[...], preferred_element_type=jnp.float32)

    @pl.when(s == 1)
    def _():
        z = _zcat()
        acc_z[...] += jnp.stack(
            [jnp.sum(z, axis=0), jnp.sum(z * z, axis=0)], axis=0)

    @pl.when((s == 2) & (i == 0))
    def _():
        st = acc_z[...]
        ssv[2:4, :] = _scale_shift(st[0], st[1], fn, g12_ref[2], g12_ref[3])
        acc_c[...] = jnp.zeros_like(acc_c)

    @pl.when(s == 2)
    def _():
        z = _zcat()
        h2 = jnp.maximum(z * ssv[2:3, :] + ssv[3:4, :], 0.0)
        c = jnp.dot(h2, wc_ref[...], preferred_element_type=jnp.float32)
        c_ref[...] = c
        acc_c[...] += jnp.stack(
            [jnp.sum(c, axis=0), jnp.sum(c * c, axis=0)], axis=0)
        st = acc_c[...]
        ss3_ref[...] = _scale_shift(st[0], st[1], fn, g3_ref[0], g3_ref[1])


def _p23(a_up, p_up, a_bd, p_bd, pc):
    n = a_up.shape[0]
    nb = n // BLK
    h = HIDDEN
    g12 = jnp.stack([
        jnp.concatenate([p_up["g1"], p_bd["g1"]]),
        jnp.concatenate([p_up["be1"], p_bd["be1"]]),
        jnp.concatenate([p_up["g2"], p_bd["g2"]]),
        jnp.concatenate([p_up["be2"], p_bd["be2"]])], axis=0)
    w2 = jnp.zeros((2 * h, 2 * h), jnp.float32)
    w2 = w2.at[:h, :h].set(p_up["W2"]).at[h:, h:].set(p_bd["W2"])
    g3 = jnp.stack([pc["g"], pc["be"]], axis=0)
    return pl.pallas_call(
        functools.partial(_p23_body, n, nb),
        grid=(3, nb),
        in_specs=[
            pl.BlockSpec((BLK, h), lambda s, i: (i, 0)),
            pl.BlockSpec((BLK, h), lambda s, i: (i, 0)),
            pl.BlockSpec((4, 2 * h), lambda s, i: (0, 0)),
            pl.BlockSpec((2 * h, 2 * h), lambda s, i: (0, 0)),
            pl.BlockSpec((2 * h, h), lambda s, i: (0, 0)),
            pl.BlockSpec((2, h), lambda s, i: (0, 0)),
        ],
        out_specs=[
            pl.BlockSpec((BLK, h), lambda s, i: (i, 0)),
            pl.BlockSpec((2, h), lambda s, i: (0, 0)),
        ],
        out_shape=[
            jax.ShapeDtypeStruct((n, h), jnp.float32),
            jax.ShapeDtypeStruct((2, h), jnp.float32),
        ],
        scratch_shapes=[
            pltpu.VMEM((2, 2 * h), jnp.float32),
            pltpu.VMEM((2, 2 * h), jnp.float32),
            pltpu.VMEM((2, h), jnp.float32),
            pltpu.VMEM((4, 2 * h), jnp.float32),
        ],
    )(a_up, a_bd, g12, w2, pc["W"], g3)


def _p4_body(nb, c_ref, ss_ref, ids_ref, out_ref, acc):
    i = pl.program_id(0)

    @pl.when(i == 0)
    def _():
        acc[...] = jnp.zeros_like(acc)

    cb = jnp.maximum(c_ref[...] * ss_ref[0:1, :] + ss_ref[1:2, :], 0.0)
    ids = ids_ref[0, 0, :]
    onehot_t = (lax.broadcasted_iota(jnp.int32, (B, BLK), 0)
                == ids[None, :]).astype(jnp.float32)
    acc[...] += jnp.dot(onehot_t, cb, preferred_element_type=jnp.float32)
    out_ref[...] = acc[...]


def _p4(c, ss, batch):
    n = c.shape[0]
    nb = n // BLK
    ids3 = batch.reshape(nb, 1, BLK)
    return pl.pallas_call(
        functools.partial(_p4_body, nb),
        grid=(nb,),
        in_specs=[
            pl.BlockSpec((BLK, HIDDEN), lambda i: (i, 0)),
            pl.BlockSpec((2, HIDDEN), lambda i: (0, 0)),
            pl.BlockSpec((1, 1, BLK), lambda i: (i, 0, 0)),
        ],
        out_specs=pl.BlockSpec((B, HIDDEN), lambda i: (0, 0)),
        out_shape=jax.ShapeDtypeStruct((B, HIDDEN), jnp.float32),
        scratch_shapes=[pltpu.VMEM((B, HIDDEN), jnp.float32)],
    )(c, ss, ids3)


def _p5_body(p0, p1, p2, w0, w1, w2, b0, b1, b2, out_ref):
    acc = jnp.zeros((B, FHM * HIDDEN), jnp.float32)
    for p, w, b in ((p0, w0, b0), (p1, w1, b1), (p2, w2, b2)):
        acc += jnp.maximum(
            jnp.dot(p[...], w[...], preferred_element_type=jnp.float32)
            + b[...], 0.0)
    out_ref[...] = acc


def _p5(pooled, lin1):
    args = list(pooled) + [lin1[d]["W"] for d in range(3)] \
        + [lin1[d]["b"].reshape(1, -1) for d in range(3)]
    return pl.pallas_call(
        _p5_body,
        out_shape=jax.ShapeDtypeStruct((B, FHM * HIDDEN), jnp.float32),
    )(*args)


_NTILES = 16
_GRP = 128
_W = 2000
_CMAX = 12480


def _seg_cfg(m):
    nch = -(-m // _CMAX)
    nch += nch % 2
    nch = max(2, nch)
    c = ((-(-m // nch)) + 15) // 16 * 16
    tail = m - (nch - 1) * c
    assert 0 < tail <= c and tail % 16 == 0 and c <= _CMAX
    return nch, c, tail


_UNROLL = 5


def _emit_segsum(core, tid, lane, table_h, src_h, dst_h, init_h, out_h,
                 win_src, win_dst, comp_src, comp_dst, rows_v0, rows_v1,
                 accum, sem0, sem1, e, m):
    nch, c, tail = _seg_cfg(m)
    npass = nch // 2
    epw = e // _NTILES
    nwin = epw // _W
    assert epw % _W == 0
    rpt, rpt_t = c // _NTILES, tail // _NTILES
    estart = tid * epw
    trash_src = lane * 8
    trash_dst = _CMAX + (lane & 7)

    for p in range(npass):
        chunk = 2 * p + core
        lo = chunk * c
        last = p == npass - 1

        if last:
            @pl.when(core == 0)
            def _():
                pltpu.sync_copy(init_h.at[pl.ds(lo + tid * rpt, rpt)],
                                accum.at[pl.ds(tid * rpt, rpt)])

            @pl.when(core == 1)
            def _():
                pltpu.sync_copy(init_h.at[pl.ds(lo + tid * rpt_t, rpt_t)],
                                accum.at[pl.ds(tid * rpt_t, rpt_t)])
        else:
            pltpu.sync_copy(init_h.at[pl.ds(lo + tid * rpt, rpt)],
                            accum.at[pl.ds(tid * rpt, rpt)])
        plsc.subcore_barrier()

        def win_body(wi, cur):
            base = estart + wi * _W
            pltpu.sync_copy(src_h.at[pl.ds(base, _W)], win_src)
            pltpu.sync_copy(dst_h.at[pl.ds(base, _W)], win_dst)

            def vbody(v, cur):
                off0 = v * (16 * _UNROLL)
                ms, cums, cnts, srcs, locs = [], [], [], [], []
                for k in range(_UNROLL):
                    off = off0 + k * 16
                    dstv = win_dst[pl.ds(off, 16)]
                    srcv = win_src[pl.ds(off, 16)]
                    local = dstv - lo
                    mask = (local >= 0) & (local < c)
                    mi = mask.astype(jnp.int32)
                    ms.append(mask)
                    cums.append(plsc.cumsum(mi))
                    cnts.append(jnp.sum(mi))
                    srcs.append(srcv)
                    locs.append(local)
                base_k = cur
                for k in range(_UNROLL):
                    pos = base_k + cums[k] - 1
                    plsc.store_scatter(comp_src, [pos], srcs[k], mask=ms[k])
                    plsc.store_scatter(
                        comp_dst, [pos >> 7, pos & 127], locs[k], mask=ms[k])
                    base_k = base_k + cnts[k]
                return base_k

            return lax.fori_loop(0, _W // (16 * _UNROLL), vbody, cur)

        cursor = lax.fori_loop(0, nwin, win_body, jnp.int32(0))

        for k in range(_GRP // 16):
            posk = cursor + k * 16 + lane
            full = posk >= 0
            plsc.store_scatter(comp_src, [posk], trash_src, mask=full)
            plsc.store_scatter(
                comp_dst, [posk >> 7, posk & 127], trash_dst, mask=full)
        ngroups = (cursor + (_GRP - 1)) >> 7

        def _issue(g, rows_v, sem):
            pltpu.async_copy(
                table_h.at[comp_src.at[pl.ds(g * _GRP, _GRP)]], rows_v, sem)

        def _wait(rows_v, sem):
            pltpu.make_async_copy(
                table_h.at[comp_src.at[pl.ds(0, _GRP)]], rows_v, sem).wait()

        @pl.when(ngroups > 0)
        def _():
            _issue(0, rows_v0, sem0)

        def gbody(k2, carry):
            g0 = k2 * 2

            @pl.when(g0 < ngroups)
            def _():
                _wait(rows_v0, sem0)

                @pl.when(g0 + 1 < ngroups)
                def _():
                    _issue(g0 + 1, rows_v1, sem1)
                pltpu.sync_copy(rows_v0, accum.at[comp_dst.at[g0]], add=True)

            @pl.when(g0 + 1 < ngroups)
            def _():
                _wait(rows_v1, sem1)

                @pl.when(g0 + 2 < ngroups)
                def _():
                    _issue(g0 + 2, rows_v0, sem0)
                pltpu.sync_copy(rows_v1, accum.at[comp_dst.at[g0 + 1]],
                                add=True)
            return carry

        lax.fori_loop(0, (ngroups + 1) >> 1, gbody, jnp.int32(0))
        plsc.subcore_barrier()

        if last:
            @pl.when(core == 0)
            def _():
                pltpu.sync_copy(accum.at[pl.ds(tid * rpt, rpt)],
                                out_h.at[pl.ds(lo + tid * rpt, rpt)])

            @pl.when(core == 1)
            def _():
                pltpu.sync_copy(accum.at[pl.ds(tid * rpt_t, rpt_t)],
                                out_h.at[pl.ds(lo + tid * rpt_t, rpt_t)])
        else:
            pltpu.sync_copy(accum.at[pl.ds(tid * rpt, rpt)],
                            out_h.at[pl.ds(lo + tid * rpt, rpt)])
        plsc.subcore_barrier()


def _layer_segsums_sc(yu0, up0s, up0d, n0, yu1, up1s, up1d, n1,
                      yb1s, b1s, b1d, yb1self, yb2s, b2s, b2d, yb2self, n2):
    h = HIDDEN
    jobs = [
        (yu0, up0s, up0d, yu0, up0s.shape[0], n0),
        (yu1, up1s, up1d, yu1, up1s.shape[0], n1),
        (yb1s, b1s, b1d, yb1self, b1s.shape[0], n1),
        (yb2s, b2s, b2d, yb2self, b2s.shape[0], n2),
    ]
    emax = max(j[4] for j in jobs)
    capg = emax // _NTILES // _GRP + 2
    cap = capg * _GRP
    mesh = plsc.VectorSubcoreMesh(core_axis_name="c", subcore_axis_name="s")

    def body(yu0_h, up0s_h, up0d_h, yu1_h, up1s_h, up1d_h,
             yb1s_h, b1s_h, b1d_h, yb1i_h, yb2s_h, b2s_h, b2d_h, yb2i_h,
             o0_h, o1_h, o2_h, o3_h,
             win_src, win_dst, comp_src, comp_dst, rows_v0, rows_v1,
             accum, sem0, sem1):
        core = lax.axis_index("c")
        tid = lax.axis_index("s")
        lane = lax.iota(jnp.int32, 16)
        tabs = (yu0_h, yu1_h, yb1s_h, yb2s_h)
        srcs = (up0s_h, up1s_h, b1s_h, b2s_h)
        dsts = (up0d_h, up1d_h, b1d_h, b2d_h)
        inits = (yu0_h, yu1_h, yb1i_h, yb2i_h)
        outs = (o0_h, o1_h, o2_h, o3_h)
        for j, (_, _, _, _, e, m) in enumerate(jobs):
            _emit_segsum(core, tid, lane, tabs[j], srcs[j], dsts[j],
                         inits[j], outs[j], win_src, win_dst, comp_src,
                         comp_dst, rows_v0, rows_v1, accum, sem0, sem1, e, m)

    run = pl.kernel(
        body,
        out_type=[jax.ShapeDtypeStruct((n0, h), jnp.float32),
                  jax.ShapeDtypeStruct((n1, h), jnp.float32),
                  jax.ShapeDtypeStruct((n1, h), jnp.float32),
                  jax.ShapeDtypeStruct((n2, h), jnp.float32)],
        mesh=mesh,
        compiler_params=pltpu.CompilerParams(
            needs_layout_passes=False, use_tc_tiling_on_sc=False),
        scratch_types=[
            pltpu.VMEM((_W,), jnp.int32),
            pltpu.VMEM((_W,), jnp.int32),
            pltpu.VMEM((cap,), jnp.int32),
            pltpu.VMEM((capg, _GRP), jnp.int32),
            pltpu.VMEM((_GRP, h), jnp.float32),
            pltpu.VMEM((_GRP, h), jnp.float32),
            pltpu.VMEM_SHARED((_CMAX + 8, h), jnp.float32),
            pltpu.SemaphoreType.DMA,
            pltpu.SemaphoreType.DMA,
        ],
    )
    return run(yu0, up0s, up0d, yu1, up1s, up1d,
               yb1s, b1s, b1d, yb1self, yb2s, b2s, b2d, yb2self)


def kernel(x0, x1, x2, up0, up1, b1_src, b1_dst, b2_src, b2_dst,
           batch0, batch1, batch2, params, lin1):
    ns = (x0.shape[0], x1.shape[0], x2.shape[0])
    srcs = [x0, x1, x2]
    sss = [None, None, None]
    for l in range(N_LAYERS):
        pl0, pl1, pl2 = params[l][0], params[l][1], params[l][2]
        yu0, yb0, yb1s = _proj(
            srcs[0], [pl0["up"]["W1"], pl0["bd"]["W1"], pl1["bd"]["W1"]], sss[0])
        yu1, yb1, yb2s = _proj(
            srcs[1], [pl1["up"]["W1"], pl1["bd"]["W1"], pl2["bd"]["W1"]], sss[1])
        yu2, yb2 = _proj(
            srcs[2], [pl2["up"]["W1"], pl2["bd"]["W1"]], sss[2])
        a_up0, a_up1, a_bd1, a_bd2 = _layer_segsums_sc(
            yu0, up0[0], up0[1], ns[0], yu1, up1[0], up1[1], ns[1],
            yb1s, b1_src, b1_dst, yb1, yb2s, b2_src, b2_dst, yb2, ns[2])
        new_srcs, new_sss = [], []
        for d, (a_up, a_bd) in enumerate(((a_up0, yb0), (a_up1, a_bd1),
                                          (yu2, a_bd2))):
            c, ss3 = _p23(a_up, params[l][d]["up"], a_bd,
                          params[l][d]["bd"], params[l][d]["comb"])
            new_srcs.append(c)
            new_sss.append(ss3)
        srcs, sss = new_srcs, new_sss
    pooled = [_p4(srcs[d], sss[d], b)
              for d, b in enumerate((batch0, batch1, batch2))]
    return _p5(pooled, lin1)


# speedup vs baseline: 3.0372x; 1.0510x over previous
import functools

import jax
import jax.numpy as jnp
from jax import lax
from jax.experimental import pallas as pl
from jax.experimental.pallas import tpu as pltpu
from jax.experimental.pallas import tpu_sc as plsc

HIDDEN = 64
IN_DIM = 128
N_LAYERS = 2
MAX_DIM = 2
B = 128
FHM = 2
EPS = 1e-5
BLK = 10000


def _scale_shift(s, ss, n, g, be):
    m = s / n
    v = ss / n - m * m
    sc = g * lax.rsqrt(v + EPS)
    sh = be - m * sc
    return jnp.stack([sc, sh], axis=0)


def _proj_body(nouts, nb, *refs):
    if len(refs) == 3 + nouts:
        x_ref, ss_ref, w_ref = refs[:3]
        xb = x_ref[...] * ss_ref[0:1, :] + ss_ref[1:2, :]
        xb = jnp.maximum(xb, 0.0)
    else:
        x_ref, w_ref = refs[:2]
        xb = x_ref[...]
    outs = refs[-nouts:]
    y = jnp.dot(xb, w_ref[...], preferred_element_type=jnp.float32)
    for k, o_ref in enumerate(outs):
        o_ref[...] = y[:, k * HIDDEN:(k + 1) * HIDDEN]


def _proj(x, ws, ss=None):
    n, k = x.shape
    nb = n // BLK
    wcat = jnp.concatenate(ws, axis=1)
    nouts = len(ws)
    in_specs = [pl.BlockSpec((BLK, k), lambda i: (i, 0))]
    args = [x]
    if ss is not None:
        in_specs.append(pl.BlockSpec((2, k), lambda i: (0, 0)))
        args.append(ss)
    in_specs.append(pl.BlockSpec((k, nouts * HIDDEN), lambda i: (0, 0)))
    args.append(wcat)
    return pl.pallas_call(
        functools.partial(_proj_body, nouts, nb),
        grid=(nb,),
        in_specs=in_specs,
        out_specs=[pl.BlockSpec((BLK, HIDDEN), lambda i: (i, 0))] * nouts,
        out_shape=[jax.ShapeDtypeStruct((n, HIDDEN), jnp.float32)] * nouts,
    )(*args)


def _p23_body(n, nb, au_ref, ab_ref, g12_ref, w2_ref, wc_ref, g3_ref,
              c_ref, ss3_ref, acc_a, acc_z, acc_c, ssv):
    s = pl.program_id(0)
    i = pl.program_id(1)
    acat = jnp.concatenate([au_ref[...], ab_ref[...]], axis=1)
    fn = float(n)

    @pl.when((s == 0) & (i == 0))
    def _():
        acc_a[...] = jnp.zeros_like(acc_a)

    @pl.when(s == 0)
    def _():
        acc_a[...] += jnp.stack(
            [jnp.sum(acat, axis=0), jnp.sum(acat * acat, axis=0)], axis=0)
        c_ref[...] = jnp.zeros_like(c_ref)
        ss3_ref[...] = jnp.zeros_like(ss3_ref)

    @pl.when((s == 1) & (i == 0))
    def _():
        st = acc_a[...]
        ssv[0:2, :] = _scale_shift(st[0], st[1], fn, g12_ref[0], g12_ref[1])
        acc_z[...] = jnp.zeros_like(acc_z)

    def _zcat():
        h1 = jnp.maximum(acat * ssv[0:1, :] + ssv[1:2, :], 0.0)
        return jnp.dot(h1, w2_ref[...], preferred_element_type=jnp.float32)

    @pl.when(s == 1)
    def _():
        z = _zcat()
        acc_z[...] += jnp.stack(
            [jnp.sum(z, axis=0), jnp.sum(z * z, axis=0)], axis=0)

    @pl.when((s == 2) & (i == 0))
    def _():
        st = acc_z[...]
        ssv[2:4, :] = _scale_shift(st[0], st[1], fn, g12_ref[2], g12_ref[3])
        acc_c[...] = jnp.zeros_like(acc_c)

    @pl.when(s == 2)
    def _():
        z = _zcat()
        h2 = jnp.maximum(z * ssv[2:3, :] + ssv[3:4, :], 0.0)
        c = jnp.dot(h2, wc_ref[...], preferred_element_type=jnp.float32)
        c_ref[...] = c
        acc_c[...] += jnp.stack(
            [jnp.sum(c, axis=0), jnp.sum(c * c, axis=0)], axis=0)
        st = acc_c[...]
        ss3_ref[...] = _scale_shift(st[0], st[1], fn, g3_ref[0], g3_ref[1])


def _p23(a_up, p_up, a_bd, p_bd, pc):
    n = a_up.shape[0]
    nb = n // BLK
    h = HIDDEN
    g12 = jnp.stack([
        jnp.concatenate([p_up["g1"], p_bd["g1"]]),
        jnp.concatenate([p_up["be1"], p_bd["be1"]]),
        jnp.concatenate([p_up["g2"], p_bd["g2"]]),
        jnp.concatenate([p_up["be2"], p_bd["be2"]])], axis=0)
    w2 = jnp.zeros((2 * h, 2 * h), jnp.float32)
    w2 = w2.at[:h, :h].set(p_up["W2"]).at[h:, h:].set(p_bd["W2"])
    g3 = jnp.stack([pc["g"], pc["be"]], axis=0)
    return pl.pallas_call(
        functools.partial(_p23_body, n, nb),
        grid=(3, nb),
        in_specs=[
            pl.BlockSpec((BLK, h), lambda s, i: (i, 0)),
            pl.BlockSpec((BLK, h), lambda s, i: (i, 0)),
            pl.BlockSpec((4, 2 * h), lambda s, i: (0, 0)),
            pl.BlockSpec((2 * h, 2 * h), lambda s, i: (0, 0)),
            pl.BlockSpec((2 * h, h), lambda s, i: (0, 0)),
            pl.BlockSpec((2, h), lambda s, i: (0, 0)),
        ],
        out_specs=[
            pl.BlockSpec((BLK, h), lambda s, i: (i, 0)),
            pl.BlockSpec((2, h), lambda s, i: (0, 0)),
        ],
        out_shape=[
            jax.ShapeDtypeStruct((n, h), jnp.float32),
            jax.ShapeDtypeStruct((2, h), jnp.float32),
        ],
        scratch_shapes=[
            pltpu.VMEM((2, 2 * h), jnp.float32),
            pltpu.VMEM((2, 2 * h), jnp.float32),
            pltpu.VMEM((2, h), jnp.float32),
            pltpu.VMEM((4, 2 * h), jnp.float32),
        ],
    )(a_up, a_bd, g12, w2, pc["W"], g3)


def _p4_body(nb, c_ref, ss_ref, ids_ref, out_ref, acc):
    i = pl.program_id(0)

    @pl.when(i == 0)
    def _():
        acc[...] = jnp.zeros_like(acc)

    cb = jnp.maximum(c_ref[...] * ss_ref[0:1, :] + ss_ref[1:2, :], 0.0)
    ids = ids_ref[0, 0, :]
    onehot_t = (lax.broadcasted_iota(jnp.int32, (B, BLK), 0)
                == ids[None, :]).astype(jnp.float32)
    acc[...] += jnp.dot(onehot_t, cb, preferred_element_type=jnp.float32)
    out_ref[...] = acc[...]


def _p4(c, ss, batch):
    n = c.shape[0]
    nb = n // BLK
    ids3 = batch.reshape(nb, 1, BLK)
    return pl.pallas_call(
        functools.partial(_p4_body, nb),
        grid=(nb,),
        in_specs=[
            pl.BlockSpec((BLK, HIDDEN), lambda i: (i, 0)),
            pl.BlockSpec((2, HIDDEN), lambda i: (0, 0)),
            pl.BlockSpec((1, 1, BLK), lambda i: (i, 0, 0)),
        ],
        out_specs=pl.BlockSpec((B, HIDDEN), lambda i: (0, 0)),
        out_shape=jax.ShapeDtypeStruct((B, HIDDEN), jnp.float32),
        scratch_shapes=[pltpu.VMEM((B, HIDDEN), jnp.float32)],
    )(c, ss, ids3)


def _p5_body(p0, p1, p2, w0, w1, w2, b0, b1, b2, out_ref):
    acc = jnp.zeros((B, FHM * HIDDEN), jnp.float32)
    for p, w, b in ((p0, w0, b0), (p1, w1, b1), (p2, w2, b2)):
        acc += jnp.maximum(
            jnp.dot(p[...], w[...], preferred_element_type=jnp.float32)
            + b[...], 0.0)
    out_ref[...] = acc


def _p5(pooled, lin1):
    args = list(pooled) + [lin1[d]["W"] for d in range(3)] \
        + [lin1[d]["b"].reshape(1, -1) for d in range(3)]
    return pl.pallas_call(
        _p5_body,
        out_shape=jax.ShapeDtypeStruct((B, FHM * HIDDEN), jnp.float32),
    )(*args)


_NTILES = 16
_GRP = 128
_W = 2000
_CMAX = 12480


def _seg_cfg(m):
    nch = -(-m // _CMAX)
    nch += nch % 2
    nch = max(2, nch)
    c = ((-(-m // nch)) + 15) // 16 * 16
    tail = m - (nch - 1) * c
    assert 0 < tail <= c and tail % 16 == 0 and c <= _CMAX
    return nch, c, tail


_UNROLL = 5


def _emit_segsum(core, tid, lane, table_h, src_h, dst_h, init_h, out_h,
                 win_src, win_dst, comp_src, comp_dst, rows_bufs,
                 accum, sems, e, m):
    nch, c, tail = _seg_cfg(m)
    npass = nch // 2
    epw = e // _NTILES
    nwin = epw // _W
    assert epw % _W == 0
    rpt, rpt_t = c // _NTILES, tail // _NTILES
    estart = tid * epw
    trash_src = lane * 8
    trash_dst = _CMAX + (lane & 7)

    def pass_body(p, carry):
        chunk = 2 * p + core
        lo = chunk * c
        is_tail = chunk == nch - 1

        @pl.when(jnp.logical_not(is_tail))
        def _():
            pltpu.sync_copy(init_h.at[pl.ds(lo + tid * rpt, rpt)],
                            accum.at[pl.ds(tid * rpt, rpt)])

        @pl.when(is_tail)
        def _():
            pltpu.sync_copy(init_h.at[pl.ds(lo + tid * rpt_t, rpt_t)],
                            accum.at[pl.ds(tid * rpt_t, rpt_t)])
        plsc.subcore_barrier()

        def win_body(wi, cur):
            base = estart + wi * _W
            pltpu.sync_copy(src_h.at[pl.ds(base, _W)], win_src)
            pltpu.sync_copy(dst_h.at[pl.ds(base, _W)], win_dst)

            def vbody(v, cur):
                off0 = v * (16 * _UNROLL)
                ms, cums, cnts, srcs, locs = [], [], [], [], []
                for k in range(_UNROLL):
                    off = off0 + k * 16
                    dstv = win_dst[pl.ds(off, 16)]
                    srcv = win_src[pl.ds(off, 16)]
                    local = dstv - lo
                    mask = (local >= 0) & (local < c)
                    mi = mask.astype(jnp.int32)
                    ms.append(mask)
                    cums.append(plsc.cumsum(mi))
                    cnts.append(jnp.sum(mi))
                    srcs.append(srcv)
                    locs.append(local)
                base_k = cur
                for k in range(_UNROLL):
                    pos = base_k + cums[k] - 1
                    plsc.store_scatter(comp_src, [pos], srcs[k], mask=ms[k])
                    plsc.store_scatter(
                        comp_dst, [pos >> 7, pos & 127], locs[k], mask=ms[k])
                    base_k = base_k + cnts[k]
                return base_k

            return lax.fori_loop(0, _W // (16 * _UNROLL), vbody, cur)

        cursor = lax.fori_loop(0, nwin, win_body, jnp.int32(0))

        for k in range(_GRP // 16):
            posk = cursor + k * 16 + lane
            full = posk >= 0
            plsc.store_scatter(comp_src, [posk], trash_src, mask=full)
            plsc.store_scatter(
                comp_dst, [posk >> 7, posk & 127], trash_dst, mask=full)
        ngroups = (cursor + (_GRP - 1)) >> 7

        nbuf = len(rows_bufs)

        def _issue(g, rows_v, sem):
            pltpu.async_copy(
                table_h.at[comp_src.at[pl.ds(g * _GRP, _GRP)]], rows_v, sem)

        def _wait(rows_v, sem):
            pltpu.make_async_copy(
                table_h.at[comp_src.at[pl.ds(0, _GRP)]], rows_v, sem).wait()

        for j in range(nbuf):
            @pl.when(j < ngroups)
            def _(j=j):
                _issue(j, rows_bufs[j], sems[j])

        def gbody(k4, carry):
            g0 = k4 * nbuf
            for j in range(nbuf):
                @pl.when(g0 + j < ngroups)
                def _(j=j):
                    _wait(rows_bufs[j], sems[j])
                    pltpu.sync_copy(rows_bufs[j],
                                    accum.at[comp_dst.at[g0 + j]], add=True)

                    @pl.when(g0 + j + nbuf < ngroups)
                    def _(j=j):
                        _issue(g0 + j + nbuf, rows_bufs[j], sems[j])
            return carry

        lax.fori_loop(0, (ngroups + nbuf - 1) >> 1, gbody, jnp.int32(0))
        plsc.subcore_barrier()

        @pl.when(jnp.logical_not(is_tail))
        def _():
            pltpu.sync_copy(accum.at[pl.ds(tid * rpt, rpt)],
                            out_h.at[pl.ds(lo + tid * rpt, rpt)])

        @pl.when(is_tail)
        def _():
            pltpu.sync_copy(accum.at[pl.ds(tid * rpt_t, rpt_t)],
                            out_h.at[pl.ds(lo + tid * rpt_t, rpt_t)])
        plsc.subcore_barrier()
        return carry

    lax.fori_loop(0, npass, pass_body, jnp.int32(0))


def _layer_segsums_sc(yu0, up0s, up0d, n0, yu1, up1s, up1d, n1,
                      yb1s, b1s, b1d, yb1self, yb2s, b2s, b2d, yb2self, n2):
    h = HIDDEN
    jobs = [
        (yu0, up0s, up0d, yu0, up0s.shape[0], n0),
        (yu1, up1s, up1d, yu1, up1s.shape[0], n1),
        (yb1s, b1s, b1d, yb1self, b1s.shape[0], n1),
        (yb2s, b2s, b2d, yb2self, b2s.shape[0], n2),
    ]
    emax = max(j[4] for j in jobs)
    capg = emax // _NTILES // _GRP + 2
    cap = capg * _GRP
    mesh = plsc.VectorSubcoreMesh(core_axis_name="c", subcore_axis_name="s")

    def body(yu0_h, up0s_h, up0d_h, yu1_h, up1s_h, up1d_h,
             yb1s_h, b1s_h, b1d_h, yb1i_h, yb2s_h, b2s_h, b2d_h, yb2i_h,
             o0_h, o1_h, o2_h, o3_h,
             win_src, win_dst, comp_src, comp_dst,
             rv0, rv1, accum, sm0, sm1):
        core = lax.axis_index("c")
        tid = lax.axis_index("s")
        lane = lax.iota(jnp.int32, 16)
        tabs = (yu0_h, yu1_h, yb1s_h, yb2s_h)
        srcs = (up0s_h, up1s_h, b1s_h, b2s_h)
        dsts = (up0d_h, up1d_h, b1d_h, b2d_h)
        inits = (yu0_h, yu1_h, yb1i_h, yb2i_h)
        outs = (o0_h, o1_h, o2_h, o3_h)
        for j, (_, _, _, _, e, m) in enumerate(jobs):
            _emit_segsum(core, tid, lane, tabs[j], srcs[j], dsts[j],
                         inits[j], outs[j], win_src, win_dst, comp_src,
                         comp_dst, [rv0, rv1], accum, [sm0, sm1], e, m)

    run = pl.kernel(
        body,
        out_type=[jax.ShapeDtypeStruct((n0, h), jnp.float32),
                  jax.ShapeDtypeStruct((n1, h), jnp.float32),
                  jax.ShapeDtypeStruct((n1, h), jnp.float32),
                  jax.ShapeDtypeStruct((n2, h), jnp.float32)],
        mesh=mesh,
        compiler_params=pltpu.CompilerParams(
            needs_layout_passes=False, use_tc_tiling_on_sc=False),
        scratch_types=[
            pltpu.VMEM((_W,), jnp.int32),
            pltpu.VMEM((_W,), jnp.int32),
            pltpu.VMEM((cap,), jnp.int32),
            pltpu.VMEM((capg, _GRP), jnp.int32),
            pltpu.VMEM((_GRP, h), jnp.float32),
            pltpu.VMEM((_GRP, h), jnp.float32),
            pltpu.VMEM_SHARED((_CMAX + 8, h), jnp.float32),
            pltpu.SemaphoreType.DMA,
            pltpu.SemaphoreType.DMA,
        ],
    )
    return run(yu0, up0s, up0d, yu1, up1s, up1d,
               yb1s, b1s, b1d, yb1self, yb2s, b2s, b2d, yb2self)


def kernel(x0, x1, x2, up0, up1, b1_src, b1_dst, b2_src, b2_dst,
           batch0, batch1, batch2, params, lin1):
    ns = (x0.shape[0], x1.shape[0], x2.shape[0])
    srcs = [x0, x1, x2]
    sss = [None, None, None]
    for l in range(N_LAYERS):
        pl0, pl1, pl2 = params[l][0], params[l][1], params[l][2]
        yu0, yb0, yb1s = _proj(
            srcs[0], [pl0["up"]["W1"], pl0["bd"]["W1"], pl1["bd"]["W1"]], sss[0])
        yu1, yb1, yb2s = _proj(
            srcs[1], [pl1["up"]["W1"], pl1["bd"]["W1"], pl2["bd"]["W1"]], sss[1])
        yu2, yb2 = _proj(
            srcs[2], [pl2["up"]["W1"], pl2["bd"]["W1"]], sss[2])
        a_up0, a_up1, a_bd1, a_bd2 = _layer_segsums_sc(
            yu0, up0[0], up0[1], ns[0], yu1, up1[0], up1[1], ns[1],
            yb1s, b1_src, b1_dst, yb1, yb2s, b2_src, b2_dst, yb2, ns[2])
        new_srcs, new_sss = [], []
        for d, (a_up, a_bd) in enumerate(((a_up0, yb0), (a_up1, a_bd1),
                                          (yu2, a_bd2))):
            c, ss3 = _p23(a_up, params[l][d]["up"], a_bd,
                          params[l][d]["bd"], params[l][d]["comb"])
            new_srcs.append(c)
            new_sss.append(ss3)
        srcs, sss = new_srcs, new_sss
    pooled = [_p4(srcs[d], sss[d], b)
              for d, b in enumerate((batch0, batch1, batch2))]
    return _p5(pooled, lin1)


# speedup vs baseline: 3.1622x; 1.0412x over previous
import functools

import jax
import jax.numpy as jnp
from jax import lax
from jax.experimental import pallas as pl
from jax.experimental.pallas import tpu as pltpu
from jax.experimental.pallas import tpu_sc as plsc

HIDDEN = 64
IN_DIM = 128
N_LAYERS = 2
MAX_DIM = 2
B = 128
FHM = 2
EPS = 1e-5
BLK = 10000


def _scale_shift(s, ss, n, g, be):
    m = s / n
    v = ss / n - m * m
    sc = g * lax.rsqrt(v + EPS)
    sh = be - m * sc
    return jnp.stack([sc, sh], axis=0)


def _proj_body(nouts, nb, *refs):
    if len(refs) == 3 + nouts:
        x_ref, ss_ref, w_ref = refs[:3]
        xb = x_ref[...] * ss_ref[0:1, :] + ss_ref[1:2, :]
        xb = jnp.maximum(xb, 0.0)
    else:
        x_ref, w_ref = refs[:2]
        xb = x_ref[...]
    outs = refs[-nouts:]
    y = jnp.dot(xb, w_ref[...], preferred_element_type=jnp.float32)
    for k, o_ref in enumerate(outs):
        o_ref[...] = y[:, k * HIDDEN:(k + 1) * HIDDEN]


def _proj(x, ws, ss=None):
    n, k = x.shape
    nb = n // BLK
    wcat = jnp.concatenate(ws, axis=1)
    nouts = len(ws)
    in_specs = [pl.BlockSpec((BLK, k), lambda i: (i, 0))]
    args = [x]
    if ss is not None:
        in_specs.append(pl.BlockSpec((2, k), lambda i: (0, 0)))
        args.append(ss)
    in_specs.append(pl.BlockSpec((k, nouts * HIDDEN), lambda i: (0, 0)))
    args.append(wcat)
    return pl.pallas_call(
        functools.partial(_proj_body, nouts, nb),
        grid=(nb,),
        in_specs=in_specs,
        out_specs=[pl.BlockSpec((BLK, HIDDEN), lambda i: (i, 0))] * nouts,
        out_shape=[jax.ShapeDtypeStruct((n, HIDDEN), jnp.float32)] * nouts,
    )(*args)


def _p23_body(n, nb, au_ref, ab_ref, g12_ref, w2_ref, wc_ref, g3_ref,
              c_ref, ss3_ref, acc_a, acc_z, acc_c, ssv):
    s = pl.program_id(0)
    i = pl.program_id(1)
    acat = jnp.concatenate([au_ref[...], ab_ref[...]], axis=1)
    fn = float(n)

    @pl.when((s == 0) & (i == 0))
    def _():
        acc_a[...] = jnp.zeros_like(acc_a)

    @pl.when(s == 0)
    def _():
        acc_a[...] += jnp.stack(
            [jnp.sum(acat, axis=0), jnp.sum(acat * acat, axis=0)], axis=0)

    @pl.when((s == 1) & (i == 0))
    def _():
        st = acc_a[...]
        ssv[0:2, :] = _scale_shift(st[0], st[1], fn, g12_ref[0], g12_ref[1])
        acc_z[...] = jnp.zeros_like(acc_z)

    def _zcat():
        h1 = jnp.maximum(acat * ssv[0:1, :] + ssv[1:2, :], 0.0)
        return jnp.dot(h1, w2_ref[...], preferred_element_type=jnp.float32)

    @pl.when(s == 1)
    def _():
        z = _zcat()
        acc_z[...] += jnp.stack(
            [jnp.sum(z, axis=0), jnp.sum(z * z, axis=0)], axis=0)

    @pl.when((s == 2) & (i == 0))
    def _():
        st = acc_z[...]
        ssv[2:4, :] = _scale_shift(st[0], st[1], fn, g12_ref[2], g12_ref[3])
        acc_c[...] = jnp.zeros_like(acc_c)

    @pl.when(s == 2)
    def _():
        z = _zcat()
        h2 = jnp.maximum(z * ssv[2:3, :] + ssv[3:4, :], 0.0)
        c = jnp.dot(h2, wc_ref[...], preferred_element_type=jnp.float32)
        c_ref[...] = c
        acc_c[...] += jnp.stack(
            [jnp.sum(c, axis=0), jnp.sum(c * c, axis=0)], axis=0)
        st = acc_c[...]
        ss3_ref[...] = _scale_shift(st[0], st[1], fn, g3_ref[0], g3_ref[1])


def _p23(a_up, p_up, a_bd, p_bd, pc):
    n = a_up.shape[0]
    nb = n // BLK
    h = HIDDEN
    g12 = jnp.stack([
        jnp.concatenate([p_up["g1"], p_bd["g1"]]),
        jnp.concatenate([p_up["be1"], p_bd["be1"]]),
        jnp.concatenate([p_up["g2"], p_bd["g2"]]),
        jnp.concatenate([p_up["be2"], p_bd["be2"]])], axis=0)
    w2 = jnp.zeros((2 * h, 2 * h), jnp.float32)
    w2 = w2.at[:h, :h].set(p_up["W2"]).at[h:, h:].set(p_bd["W2"])
    g3 = jnp.stack([pc["g"], pc["be"]], axis=0)
    return pl.pallas_call(
        functools.partial(_p23_body, n, nb),
        grid=(3, nb),
        in_specs=[
            pl.BlockSpec((BLK, h), lambda s, i: (i, 0)),
            pl.BlockSpec((BLK, h), lambda s, i: (i, 0)),
            pl.BlockSpec((4, 2 * h), lambda s, i: (0, 0)),
            pl.BlockSpec((2 * h, 2 * h), lambda s, i: (0, 0)),
            pl.BlockSpec((2 * h, h), lambda s, i: (0, 0)),
            pl.BlockSpec((2, h), lambda s, i: (0, 0)),
        ],
        out_specs=[
            pl.BlockSpec((BLK, h), lambda s, i: (jnp.where(s == 2, i, 0), 0)),
            pl.BlockSpec((2, h), lambda s, i: (0, 0)),
        ],
        out_shape=[
            jax.ShapeDtypeStruct((n, h), jnp.float32),
            jax.ShapeDtypeStruct((2, h), jnp.float32),
        ],
        scratch_shapes=[
            pltpu.VMEM((2, 2 * h), jnp.float32),
            pltpu.VMEM((2, 2 * h), jnp.float32),
            pltpu.VMEM((2, h), jnp.float32),
            pltpu.VMEM((4, 2 * h), jnp.float32),
        ],
    )(a_up, a_bd, g12, w2, pc["W"], g3)


def _p4_body(nb, c_ref, ss_ref, ids_ref, out_ref, acc):
    i = pl.program_id(0)

    @pl.when(i == 0)
    def _():
        acc[...] = jnp.zeros_like(acc)

    cb = jnp.maximum(c_ref[...] * ss_ref[0:1, :] + ss_ref[1:2, :], 0.0)
    ids = ids_ref[0, 0, :]
    onehot_t = (lax.broadcasted_iota(jnp.int32, (B, BLK), 0)
                == ids[None, :]).astype(jnp.float32)
    acc[...] += jnp.dot(onehot_t, cb, preferred_element_type=jnp.float32)
    out_ref[...] = acc[...]


def _p4(c, ss, batch):
    n = c.shape[0]
    nb = n // BLK
    ids3 = batch.reshape(nb, 1, BLK)
    return pl.pallas_call(
        functools.partial(_p4_body, nb),
        grid=(nb,),
        in_specs=[
            pl.BlockSpec((BLK, HIDDEN), lambda i: (i, 0)),
            pl.BlockSpec((2, HIDDEN), lambda i: (0, 0)),
            pl.BlockSpec((1, 1, BLK), lambda i: (i, 0, 0)),
        ],
        out_specs=pl.BlockSpec((B, HIDDEN), lambda i: (0, 0)),
        out_shape=jax.ShapeDtypeStruct((B, HIDDEN), jnp.float32),
        scratch_shapes=[pltpu.VMEM((B, HIDDEN), jnp.float32)],
    )(c, ss, ids3)


def _p5_body(p0, p1, p2, w0, w1, w2, b0, b1, b2, out_ref):
    acc = jnp.zeros((B, FHM * HIDDEN), jnp.float32)
    for p, w, b in ((p0, w0, b0), (p1, w1, b1), (p2, w2, b2)):
        acc += jnp.maximum(
            jnp.dot(p[...], w[...], preferred_element_type=jnp.float32)
            + b[...], 0.0)
    out_ref[...] = acc


def _p5(pooled, lin1):
    args = list(pooled) + [lin1[d]["W"] for d in range(3)] \
        + [lin1[d]["b"].reshape(1, -1) for d in range(3)]
    return pl.pallas_call(
        _p5_body,
        out_shape=jax.ShapeDtypeStruct((B, FHM * HIDDEN), jnp.float32),
    )(*args)


_NTILES = 16
_GRP = 128
_W = 2000
_CMAX = 12480


def _seg_cfg(m):
    nch = -(-m // _CMAX)
    nch += nch % 2
    nch = max(2, nch)
    c = ((-(-m // nch)) + 15) // 16 * 16
    tail = m - (nch - 1) * c
    assert 0 < tail <= c and tail % 16 == 0 and c <= _CMAX
    return nch, c, tail


_UNROLL = 5


def _emit_segsum(core, tid, lane, table_h, src_h, dst_h, init_h, out_h,
                 win_src, win_dst, comp_src, comp_dst, rows_bufs,
                 accum, sems, e, m):
    nch, c, tail = _seg_cfg(m)
    npass = nch // 2
    epw = e // _NTILES
    nwin = epw // _W
    assert epw % _W == 0
    rpt, rpt_t = c // _NTILES, tail // _NTILES
    estart = tid * epw
    trash_src = lane * 8
    trash_dst = _CMAX + (lane & 7)

    def pass_body(p, carry):
        chunk = 2 * p + core
        lo = chunk * c
        is_tail = chunk == nch - 1

        @pl.when(jnp.logical_not(is_tail))
        def _():
            pltpu.sync_copy(init_h.at[pl.ds(lo + tid * rpt, rpt)],
                            accum.at[pl.ds(tid * rpt, rpt)])

        @pl.when(is_tail)
        def _():
            pltpu.sync_copy(init_h.at[pl.ds(lo + tid * rpt_t, rpt_t)],
                            accum.at[pl.ds(tid * rpt_t, rpt_t)])
        plsc.subcore_barrier()

        def win_body(wi, cur):
            base = estart + wi * _W
            pltpu.sync_copy(src_h.at[pl.ds(base, _W)], win_src)
            pltpu.sync_copy(dst_h.at[pl.ds(base, _W)], win_dst)

            def vbody(v, cur):
                off0 = v * (16 * _UNROLL)
                ms, cums, cnts, srcs, locs = [], [], [], [], []
                for k in range(_UNROLL):
                    off = off0 + k * 16
                    dstv = win_dst[pl.ds(off, 16)]
                    srcv = win_src[pl.ds(off, 16)]
                    local = dstv - lo
                    mask = (local >= 0) & (local < c)
                    mi = mask.astype(jnp.int32)
                    ms.append(mask)
                    cums.append(plsc.cumsum(mi))
                    cnts.append(jnp.sum(mi))
                    srcs.append(srcv)
                    locs.append(local)
                base_k = cur
                for k in range(_UNROLL):
                    pos = base_k + cums[k] - 1
                    plsc.store_scatter(comp_src, [pos], srcs[k], mask=ms[k])
                    plsc.store_scatter(
                        comp_dst, [pos >> 7, pos & 127], locs[k], mask=ms[k])
                    base_k = base_k + cnts[k]
                return base_k

            return lax.fori_loop(0, _W // (16 * _UNROLL), vbody, cur)

        cursor = lax.fori_loop(0, nwin, win_body, jnp.int32(0))

        for k in range(_GRP // 16):
            posk = cursor + k * 16 + lane
            full = posk >= 0
            plsc.store_scatter(comp_src, [posk], trash_src, mask=full)
            plsc.store_scatter(
                comp_dst, [posk >> 7, posk & 127], trash_dst, mask=full)
        ngroups = (cursor + (_GRP - 1)) >> 7

        nbuf = len(rows_bufs)

        def _issue(g, rows_v, sem):
            pltpu.async_copy(
                table_h.at[comp_src.at[pl.ds(g * _GRP, _GRP)]], rows_v, sem)

        def _wait(rows_v, sem):
            pltpu.make_async_copy(
                table_h.at[comp_src.at[pl.ds(0, _GRP)]], rows_v, sem).wait()

        for j in range(nbuf):
            @pl.when(j < ngroups)
            def _(j=j):
                _issue(j, rows_bufs[j], sems[j])

        def gbody(k4, carry):
            g0 = k4 * nbuf
            for j in range(nbuf):
                @pl.when(g0 + j < ngroups)
                def _(j=j):
                    _wait(rows_bufs[j], sems[j])
                    pltpu.sync_copy(rows_bufs[j],
                                    accum.at[comp_dst.at[g0 + j]], add=True)

                    @pl.when(g0 + j + nbuf < ngroups)
                    def _(j=j):
                        _issue(g0 + j + nbuf, rows_bufs[j], sems[j])
            return carry

        lax.fori_loop(0, (ngroups + nbuf - 1) >> 1, gbody, jnp.int32(0))
        plsc.subcore_barrier()

        @pl.when(jnp.logical_not(is_tail))
        def _():
            pltpu.sync_copy(accum.at[pl.ds(tid * rpt, rpt)],
                            out_h.at[pl.ds(lo + tid * rpt, rpt)])

        @pl.when(is_tail)
        def _():
            pltpu.sync_copy(accum.at[pl.ds(tid * rpt_t, rpt_t)],
                            out_h.at[pl.ds(lo + tid * rpt_t, rpt_t)])
        plsc.subcore_barrier()
        return carry

    lax.fori_loop(0, npass, pass_body, jnp.int32(0))


def _layer_segsums_sc(yu0, up0s, up0d, n0, yu1, up1s, up1d, n1,
                      yb1s, b1s, b1d, yb1self, yb2s, b2s, b2d, yb2self, n2):
    h = HIDDEN
    jobs = [
        (yu0, up0s, up0d, yu0, up0s.shape[0], n0),
        (yu1, up1s, up1d, yu1, up1s.shape[0], n1),
        (yb1s, b1s, b1d, yb1self, b1s.shape[0], n1),
        (yb2s, b2s, b2d, yb2self, b2s.shape[0], n2),
    ]
    emax = max(j[4] for j in jobs)
    capg = emax // _NTILES // _GRP + 2
    cap = capg * _GRP
    mesh = plsc.VectorSubcoreMesh(core_axis_name="c", subcore_axis_name="s")

    def body(yu0_h, up0s_h, up0d_h, yu1_h, up1s_h, up1d_h,
             yb1s_h, b1s_h, b1d_h, yb1i_h, yb2s_h, b2s_h, b2d_h, yb2i_h,
             o0_h, o1_h, o2_h, o3_h,
             win_src, win_dst, comp_src, comp_dst,
             rv0, rv1, accum, sm0, sm1):
        core = lax.axis_index("c")
        tid = lax.axis_index("s")
        lane = lax.iota(jnp.int32, 16)
        tabs = (yu0_h, yu1_h, yb1s_h, yb2s_h)
        srcs = (up0s_h, up1s_h, b1s_h, b2s_h)
        dsts = (up0d_h, up1d_h, b1d_h, b2d_h)
        inits = (yu0_h, yu1_h, yb1i_h, yb2i_h)
        outs = (o0_h, o1_h, o2_h, o3_h)
        for j, (_, _, _, _, e, m) in enumerate(jobs):
            _emit_segsum(core, tid, lane, tabs[j], srcs[j], dsts[j],
                         inits[j], outs[j], win_src, win_dst, comp_src,
                         comp_dst, [rv0, rv1], accum, [sm0, sm1], e, m)

    run = pl.kernel(
        body,
        out_type=[jax.ShapeDtypeStruct((n0, h), jnp.float32),
                  jax.ShapeDtypeStruct((n1, h), jnp.float32),
                  jax.ShapeDtypeStruct((n1, h), jnp.float32),
                  jax.ShapeDtypeStruct((n2, h), jnp.float32)],
        mesh=mesh,
        compiler_params=pltpu.CompilerParams(
            needs_layout_passes=False, use_tc_tiling_on_sc=False),
        scratch_types=[
            pltpu.VMEM((_W,), jnp.int32),
            pltpu.VMEM((_W,), jnp.int32),
            pltpu.VMEM((cap,), jnp.int32),
            pltpu.VMEM((capg, _GRP), jnp.int32),
            pltpu.VMEM((_GRP, h), jnp.float32),
            pltpu.VMEM((_GRP, h), jnp.float32),
            pltpu.VMEM_SHARED((_CMAX + 8, h), jnp.float32),
            pltpu.SemaphoreType.DMA,
            pltpu.SemaphoreType.DMA,
        ],
    )
    return run(yu0, up0s, up0d, yu1, up1s, up1d,
               yb1s, b1s, b1d, yb1self, yb2s, b2s, b2d, yb2self)


def kernel(x0, x1, x2, up0, up1, b1_src, b1_dst, b2_src, b2_dst,
           batch0, batch1, batch2, params, lin1):
    ns = (x0.shape[0], x1.shape[0], x2.shape[0])
    srcs = [x0, x1, x2]
    sss = [None, None, None]
    for l in range(N_LAYERS):
        pl0, pl1, pl2 = params[l][0], params[l][1], params[l][2]
        yu0, yb0, yb1s = _proj(
            srcs[0], [pl0["up"]["W1"], pl0["bd"]["W1"], pl1["bd"]["W1"]], sss[0])
        yu1, yb1, yb2s = _proj(
            srcs[1], [pl1["up"]["W1"], pl1["bd"]["W1"], pl2["bd"]["W1"]], sss[1])
        yu2, yb2 = _proj(
            srcs[2], [pl2["up"]["W1"], pl2["bd"]["W1"]], sss[2])
        a_up0, a_up1, a_bd1, a_bd2 = _layer_segsums_sc(
            yu0, up0[0], up0[1], ns[0], yu1, up1[0], up1[1], ns[1],
            yb1s, b1_src, b1_dst, yb1, yb2s, b2_src, b2_dst, yb2, ns[2])
        new_srcs, new_sss = [], []
        for d, (a_up, a_bd) in enumerate(((a_up0, yb0), (a_up1, a_bd1),
                                          (yu2, a_bd2))):
            c, ss3 = _p23(a_up, params[l][d]["up"], a_bd,
                          params[l][d]["bd"], params[l][d]["comb"])
            new_srcs.append(c)
            new_sss.append(ss3)
        srcs, sss = new_srcs, new_sss
    pooled = [_p4(srcs[d], sss[d], b)
              for d, b in enumerate((batch0, batch1, batch2))]
    return _p5(pooled, lin1)


# speedup vs baseline: 3.2624x; 1.0317x over previous
import functools

import jax
import jax.numpy as jnp
from jax import lax
from jax.experimental import pallas as pl
from jax.experimental.pallas import tpu as pltpu
from jax.experimental.pallas import tpu_sc as plsc

HIDDEN = 64
IN_DIM = 128
N_LAYERS = 2
MAX_DIM = 2
B = 128
FHM = 2
EPS = 1e-5
BLK = 10000


def _scale_shift(s, ss, n, g, be):
    m = s / n
    v = ss / n - m * m
    sc = g * lax.rsqrt(v + EPS)
    sh = be - m * sc
    return jnp.stack([sc, sh], axis=0)


def _proj_body(nouts, nb, *refs):
    if len(refs) == 3 + nouts:
        x_ref, ss_ref, w_ref = refs[:3]
        xb = x_ref[...] * ss_ref[0:1, :] + ss_ref[1:2, :]
        xb = jnp.maximum(xb, 0.0)
    else:
        x_ref, w_ref = refs[:2]
        xb = x_ref[...]
    outs = refs[-nouts:]
    y = jnp.dot(xb, w_ref[...], preferred_element_type=jnp.float32)
    for k, o_ref in enumerate(outs):
        o_ref[...] = y[:, k * HIDDEN:(k + 1) * HIDDEN]


def _proj(x, ws, ss=None):
    n, k = x.shape
    nb = n // BLK
    wcat = jnp.concatenate(ws, axis=1)
    nouts = len(ws)
    in_specs = [pl.BlockSpec((BLK, k), lambda i: (i, 0))]
    args = [x]
    if ss is not None:
        in_specs.append(pl.BlockSpec((2, k), lambda i: (0, 0)))
        args.append(ss)
    in_specs.append(pl.BlockSpec((k, nouts * HIDDEN), lambda i: (0, 0)))
    args.append(wcat)
    return pl.pallas_call(
        functools.partial(_proj_body, nouts, nb),
        grid=(nb,),
        in_specs=in_specs,
        out_specs=[pl.BlockSpec((BLK, HIDDEN), lambda i: (i, 0))] * nouts,
        out_shape=[jax.ShapeDtypeStruct((n, HIDDEN), jnp.float32)] * nouts,
    )(*args)


def _p23_body(n, nb, au_ref, ab_ref, g12_ref, w2_ref, wc_ref, g3_ref,
              c_ref, ss3_ref, acc_a, acc_z, acc_c, ssv):
    s = pl.program_id(0)
    i = pl.program_id(1)
    acat = jnp.concatenate([au_ref[...], ab_ref[...]], axis=1)
    fn = float(n)

    @pl.when((s == 0) & (i == 0))
    def _():
        acc_a[...] = jnp.zeros_like(acc_a)

    @pl.when(s == 0)
    def _():
        acc_a[...] += jnp.stack(
            [jnp.sum(acat, axis=0), jnp.sum(acat * acat, axis=0)], axis=0)

    @pl.when((s == 1) & (i == 0))
    def _():
        st = acc_a[...]
        ssv[0:2, :] = _scale_shift(st[0], st[1], fn, g12_ref[0], g12_ref[1])
        acc_z[...] = jnp.zeros_like(acc_z)

    def _zcat():
        h1 = jnp.maximum(acat * ssv[0:1, :] + ssv[1:2, :], 0.0)
        return jnp.dot(h1, w2_ref[...], preferred_element_type=jnp.float32)

    @pl.when(s == 1)
    def _():
        z = _zcat()
        acc_z[...] += jnp.stack(
            [jnp.sum(z, axis=0), jnp.sum(z * z, axis=0)], axis=0)

    @pl.when((s == 2) & (i == 0))
    def _():
        st = acc_z[...]
        ssv[2:4, :] = _scale_shift(st[0], st[1], fn, g12_ref[2], g12_ref[3])
        acc_c[...] = jnp.zeros_like(acc_c)

    @pl.when(s == 2)
    def _():
        z = _zcat()
        h2 = jnp.maximum(z * ssv[2:3, :] + ssv[3:4, :], 0.0)
        c = jnp.dot(h2, wc_ref[...], preferred_element_type=jnp.float32)
        c_ref[...] = c
        acc_c[...] += jnp.stack(
            [jnp.sum(c, axis=0), jnp.sum(c * c, axis=0)], axis=0)
        st = acc_c[...]
        ss3_ref[...] = _scale_shift(st[0], st[1], fn, g3_ref[0], g3_ref[1])


def _p23(a_up, p_up, a_bd, p_bd, pc):
    n = a_up.shape[0]
    nb = n // BLK
    h = HIDDEN
    g12 = jnp.stack([
        jnp.concatenate([p_up["g1"], p_bd["g1"]]),
        jnp.concatenate([p_up["be1"], p_bd["be1"]]),
        jnp.concatenate([p_up["g2"], p_bd["g2"]]),
        jnp.concatenate([p_up["be2"], p_bd["be2"]])], axis=0)
    w2 = jnp.zeros((2 * h, 2 * h), jnp.float32)
    w2 = w2.at[:h, :h].set(p_up["W2"]).at[h:, h:].set(p_bd["W2"])
    g3 = jnp.stack([pc["g"], pc["be"]], axis=0)
    return pl.pallas_call(
        functools.partial(_p23_body, n, nb),
        grid=(3, nb),
        in_specs=[
            pl.BlockSpec((BLK, h), lambda s, i: (i, 0)),
            pl.BlockSpec((BLK, h), lambda s, i: (i, 0)),
            pl.BlockSpec((4, 2 * h), lambda s, i: (0, 0)),
            pl.BlockSpec((2 * h, 2 * h), lambda s, i: (0, 0)),
            pl.BlockSpec((2 * h, h), lambda s, i: (0, 0)),
            pl.BlockSpec((2, h), lambda s, i: (0, 0)),
        ],
        out_specs=[
            pl.BlockSpec((BLK, h), lambda s, i: (jnp.where(s == 2, i, 0), 0)),
            pl.BlockSpec((2, h), lambda s, i: (0, 0)),
        ],
        out_shape=[
            jax.ShapeDtypeStruct((n, h), jnp.float32),
            jax.ShapeDtypeStruct((2, h), jnp.float32),
        ],
        scratch_shapes=[
            pltpu.VMEM((2, 2 * h), jnp.float32),
            pltpu.VMEM((2, 2 * h), jnp.float32),
            pltpu.VMEM((2, h), jnp.float32),
            pltpu.VMEM((4, 2 * h), jnp.float32),
        ],
    )(a_up, a_bd, g12, w2, pc["W"], g3)


def _p4_body(nb, c_ref, ss_ref, ids_ref, out_ref, acc):
    i = pl.program_id(0)

    @pl.when(i == 0)
    def _():
        acc[...] = jnp.zeros_like(acc)

    cb = jnp.maximum(c_ref[...] * ss_ref[0:1, :] + ss_ref[1:2, :], 0.0)
    ids = ids_ref[0, 0, :]
    onehot_t = (lax.broadcasted_iota(jnp.int32, (B, BLK), 0)
                == ids[None, :]).astype(jnp.float32)
    acc[...] += jnp.dot(onehot_t, cb, preferred_element_type=jnp.float32)
    out_ref[...] = acc[...]


def _p4(c, ss, batch):
    n = c.shape[0]
    nb = n // BLK
    ids3 = batch.reshape(nb, 1, BLK)
    return pl.pallas_call(
        functools.partial(_p4_body, nb),
        grid=(nb,),
        in_specs=[
            pl.BlockSpec((BLK, HIDDEN), lambda i: (i, 0)),
            pl.BlockSpec((2, HIDDEN), lambda i: (0, 0)),
            pl.BlockSpec((1, 1, BLK), lambda i: (i, 0, 0)),
        ],
        out_specs=pl.BlockSpec((B, HIDDEN), lambda i: (0, 0)),
        out_shape=jax.ShapeDtypeStruct((B, HIDDEN), jnp.float32),
        scratch_shapes=[pltpu.VMEM((B, HIDDEN), jnp.float32)],
    )(c, ss, ids3)


def _p5_body(p0, p1, p2, w0, w1, w2, b0, b1, b2, out_ref):
    acc = jnp.zeros((B, FHM * HIDDEN), jnp.float32)
    for p, w, b in ((p0, w0, b0), (p1, w1, b1), (p2, w2, b2)):
        acc += jnp.maximum(
            jnp.dot(p[...], w[...], preferred_element_type=jnp.float32)
            + b[...], 0.0)
    out_ref[...] = acc


def _p5(pooled, lin1):
    args = list(pooled) + [lin1[d]["W"] for d in range(3)] \
        + [lin1[d]["b"].reshape(1, -1) for d in range(3)]
    return pl.pallas_call(
        _p5_body,
        out_shape=jax.ShapeDtypeStruct((B, FHM * HIDDEN), jnp.float32),
    )(*args)


_NTILES = 16
_GRP = 128
_W = 2000
_CMAX = 12480


def _seg_cfg(m):
    nch = -(-m // _CMAX)
    nch += nch % 2
    nch = max(2, nch)
    c = ((-(-m // nch)) + 15) // 16 * 16
    tail = m - (nch - 1) * c
    assert 0 < tail <= c and tail % 16 == 0 and c <= _CMAX
    return nch, c, tail


_UNROLL = 5


def _emit_segsum(core, tid, lane, table_h, src_h, dst_h, init_h, out_h,
                 win_src, win_dst, comp_src, comp_dst, rows_bufs,
                 accum, sems, e, m):
    nch, c, tail = _seg_cfg(m)
    npass = nch // 2
    epw = e // _NTILES
    nwin = epw // _W
    assert epw % _W == 0
    rpt, rpt_t = c // _NTILES, tail // _NTILES
    estart = tid * epw
    trash_src = lane * 8
    trash_dst = _CMAX + (lane & 7)

    def pass_body(p, carry):
        chunk = 2 * p + core
        lo = chunk * c
        is_tail = chunk == nch - 1

        @pl.when(jnp.logical_not(is_tail))
        def _():
            pltpu.sync_copy(init_h.at[pl.ds(lo + tid * rpt, rpt)],
                            accum.at[pl.ds(tid * rpt, rpt)])

        @pl.when(is_tail)
        def _():
            pltpu.sync_copy(init_h.at[pl.ds(lo + tid * rpt_t, rpt_t)],
                            accum.at[pl.ds(tid * rpt_t, rpt_t)])
        plsc.subcore_barrier()

        def win_body(wi, cur):
            base = estart + wi * _W
            pltpu.sync_copy(src_h.at[pl.ds(base, _W)], win_src)
            pltpu.sync_copy(dst_h.at[pl.ds(base, _W)], win_dst)

            def vbody(v, cur_v):
                off0 = v * (16 * _UNROLL)
                ms, cums, cnts, srcs, locs = [], [], [], [], []
                for k in range(_UNROLL):
                    off = off0 + k * 16
                    dstv = win_dst[pl.ds(off, 16)]
                    srcv = win_src[pl.ds(off, 16)]
                    local = dstv - lo
                    mask = (local >= 0) & (local < c)
                    mi = mask.astype(jnp.int32)
                    ms.append(mask)
                    cums.append(plsc.cumsum(mi))
                    cnts.append(plsc.all_reduce_population_count(mask))
                    srcs.append(srcv)
                    locs.append(local)
                base_v = cur_v
                for k in range(_UNROLL):
                    pos = base_v + cums[k] - 1
                    plsc.store_scatter(comp_src, [pos], srcs[k], mask=ms[k])
                    plsc.store_scatter(
                        comp_dst, [pos >> 7, pos & 127], locs[k], mask=ms[k])
                    base_v = base_v + cnts[k]
                return base_v

            return lax.fori_loop(0, _W // (16 * _UNROLL), vbody, cur)

        cursor_v = lax.fori_loop(0, nwin, win_body,
                                 jnp.zeros((16,), jnp.int32))
        cursor = jnp.max(cursor_v)

        for k in range(_GRP // 16):
            posk = cursor + k * 16 + lane
            full = posk >= 0
            plsc.store_scatter(comp_src, [posk], trash_src, mask=full)
            plsc.store_scatter(
                comp_dst, [posk >> 7, posk & 127], trash_dst, mask=full)
        ngroups = (cursor + (_GRP - 1)) >> 7

        nbuf = len(rows_bufs)

        def _issue(g, rows_v, sem):
            pltpu.async_copy(
                table_h.at[comp_src.at[pl.ds(g * _GRP, _GRP)]], rows_v, sem)

        def _wait(rows_v, sem):
            pltpu.make_async_copy(
                table_h.at[comp_src.at[pl.ds(0, _GRP)]], rows_v, sem).wait()

        for j in range(nbuf):
            @pl.when(j < ngroups)
            def _(j=j):
                _issue(j, rows_bufs[j], sems[j])

        def gbody(k4, carry):
            g0 = k4 * nbuf
            for j in range(nbuf):
                @pl.when(g0 + j < ngroups)
                def _(j=j):
                    _wait(rows_bufs[j], sems[j])
                    pltpu.sync_copy(rows_bufs[j],
                                    accum.at[comp_dst.at[g0 + j]], add=True)

                    @pl.when(g0 + j + nbuf < ngroups)
                    def _(j=j):
                        _issue(g0 + j + nbuf, rows_bufs[j], sems[j])
            return carry

        lax.fori_loop(0, (ngroups + nbuf - 1) >> 1, gbody, jnp.int32(0))
        plsc.subcore_barrier()

        @pl.when(jnp.logical_not(is_tail))
        def _():
            pltpu.sync_copy(accum.at[pl.ds(tid * rpt, rpt)],
                            out_h.at[pl.ds(lo + tid * rpt, rpt)])

        @pl.when(is_tail)
        def _():
            pltpu.sync_copy(accum.at[pl.ds(tid * rpt_t, rpt_t)],
                            out_h.at[pl.ds(lo + tid * rpt_t, rpt_t)])
        plsc.subcore_barrier()
        return carry

    lax.fori_loop(0, npass, pass_body, jnp.int32(0))


def _layer_segsums_sc(yu0, up0s, up0d, n0, yu1, up1s, up1d, n1,
                      yb1s, b1s, b1d, yb1self, yb2s, b2s, b2d, yb2self, n2):
    h = HIDDEN
    jobs = [
        (yu0, up0s, up0d, yu0, up0s.shape[0], n0),
        (yu1, up1s, up1d, yu1, up1s.shape[0], n1),
        (yb1s, b1s, b1d, yb1self, b1s.shape[0], n1),
        (yb2s, b2s, b2d, yb2self, b2s.shape[0], n2),
    ]
    emax = max(j[4] for j in jobs)
    capg = emax // _NTILES // _GRP + 2
    cap = capg * _GRP
    mesh = plsc.VectorSubcoreMesh(core_axis_name="c", subcore_axis_name="s")

    def body(yu0_h, up0s_h, up0d_h, yu1_h, up1s_h, up1d_h,
             yb1s_h, b1s_h, b1d_h, yb1i_h, yb2s_h, b2s_h, b2d_h, yb2i_h,
             o0_h, o1_h, o2_h, o3_h,
             win_src, win_dst, comp_src, comp_dst,
             rv0, rv1, accum, sm0, sm1):
        core = lax.axis_index("c")
        tid = lax.axis_index("s")
        lane = lax.iota(jnp.int32, 16)
        tabs = (yu0_h, yu1_h, yb1s_h, yb2s_h)
        srcs = (up0s_h, up1s_h, b1s_h, b2s_h)
        dsts = (up0d_h, up1d_h, b1d_h, b2d_h)
        inits = (yu0_h, yu1_h, yb1i_h, yb2i_h)
        outs = (o0_h, o1_h, o2_h, o3_h)
        for j, (_, _, _, _, e, m) in enumerate(jobs):
            _emit_segsum(core, tid, lane, tabs[j], srcs[j], dsts[j],
                         inits[j], outs[j], win_src, win_dst, comp_src,
                         comp_dst, [rv0, rv1], accum, [sm0, sm1], e, m)

    run = pl.kernel(
        body,
        out_type=[jax.ShapeDtypeStruct((n0, h), jnp.float32),
                  jax.ShapeDtypeStruct((n1, h), jnp.float32),
                  jax.ShapeDtypeStruct((n1, h), jnp.float32),
                  jax.ShapeDtypeStruct((n2, h), jnp.float32)],
        mesh=mesh,
        compiler_params=pltpu.CompilerParams(
            needs_layout_passes=False, use_tc_tiling_on_sc=False),
        scratch_types=[
            pltpu.VMEM((_W,), jnp.int32),
            pltpu.VMEM((_W,), jnp.int32),
            pltpu.VMEM((cap,), jnp.int32),
            pltpu.VMEM((capg, _GRP), jnp.int32),
            pltpu.VMEM((_GRP, h), jnp.float32),
            pltpu.VMEM((_GRP, h), jnp.float32),
            pltpu.VMEM_SHARED((_CMAX + 8, h), jnp.float32),
            pltpu.SemaphoreType.DMA,
            pltpu.SemaphoreType.DMA,
        ],
    )
    return run(yu0, up0s, up0d, yu1, up1s, up1d,
               yb1s, b1s, b1d, yb1self, yb2s, b2s, b2d, yb2self)


def kernel(x0, x1, x2, up0, up1, b1_src, b1_dst, b2_src, b2_dst,
           batch0, batch1, batch2, params, lin1):
    ns = (x0.shape[0], x1.shape[0], x2.shape[0])
    srcs = [x0, x1, x2]
    sss = [None, None, None]
    for l in range(N_LAYERS):
        pl0, pl1, pl2 = params[l][0], params[l][1], params[l][2]
        yu0, yb0, yb1s = _proj(
            srcs[0], [pl0["up"]["W1"], pl0["bd"]["W1"], pl1["bd"]["W1"]], sss[0])
        yu1, yb1, yb2s = _proj(
            srcs[1], [pl1["up"]["W1"], pl1["bd"]["W1"], pl2["bd"]["W1"]], sss[1])
        yu2, yb2 = _proj(
            srcs[2], [pl2["up"]["W1"], pl2["bd"]["W1"]], sss[2])
        a_up0, a_up1, a_bd1, a_bd2 = _layer_segsums_sc(
            yu0, up0[0], up0[1], ns[0], yu1, up1[0], up1[1], ns[1],
            yb1s, b1_src, b1_dst, yb1, yb2s, b2_src, b2_dst, yb2, ns[2])
        new_srcs, new_sss = [], []
        for d, (a_up, a_bd) in enumerate(((a_up0, yb0), (a_up1, a_bd1),
                                          (yu2, a_bd2))):
            c, ss3 = _p23(a_up, params[l][d]["up"], a_bd,
                          params[l][d]["bd"], params[l][d]["comb"])
            new_srcs.append(c)
            new_sss.append(ss3)
        srcs, sss = new_srcs, new_sss
    pooled = [_p4(srcs[d], sss[d], b)
              for d, b in enumerate((batch0, batch1, batch2))]
    return _p5(pooled, lin1)
